# Initial kernel scaffold; baseline (speedup 1.0000x reference)
import jax
import jax.numpy as jnp
from jax.experimental import pallas as pl


def kernel(nodes, x, edges, virtual, edge_index, node_batch, lengths, t, cond, params):
    raise NotImplementedError("write your pallas kernel here")


# speedup vs baseline: 4.3005x; 4.3005x over previous
import functools

import jax
import jax.numpy as jnp
from jax import lax
from jax.experimental import pallas as pl
from jax.experimental.pallas import tpu as pltpu
from jax.experimental.pallas import tpu_sc as plsc

NB = 16
NC, NS = 2, 16
NW = NC * NS
CHUNK = 128


def _silu(x):
    return x * jax.nn.sigmoid(x)


def _dot(a, b):
    return jnp.dot(a, b, preferred_element_type=jnp.float32)


def _dotT(a, b):
    return lax.dot_general(a, b, (((0,), (0,)), ((), ())),
                           preferred_element_type=jnp.float32)


def _full(shape):
    return pl.BlockSpec(shape, lambda i: (0,) * len(shape))


def _rows(r, c):
    return pl.BlockSpec((r, c), lambda i: (i, 0))


def _pre_body(nodes_ref, xp_ref, nb_ref, wn_ref, wx_ref, b1_ref, w2_ref,
              b2_ref, ws_ref, wd_ref, be_ref, h_ref, ts_ref, td_ref, oh_ref):
    r = nodes_ref.shape[0]
    hmid = _silu(_dot(nodes_ref[...], wn_ref[...])
                 + _dot(xp_ref[...], wx_ref[...]) + b1_ref[...])
    h = _dot(hmid, w2_ref[...]) + b2_ref[...]
    h_ref[...] = h
    cols = lax.broadcasted_iota(jnp.int32, (r, NB), 1)
    oh = (nb_ref[...] == cols).astype(jnp.float32)
    oh_ref[...] = oh
    z = jnp.zeros((r, 128 - 64 - NB), jnp.float32)
    ts_ref[...] = jnp.concatenate(
        [_dot(h, ws_ref[...]) + be_ref[...], oh, z], axis=1)
    td_ref[...] = jnp.concatenate(
        [_dot(h, wd_ref[...]), jnp.zeros((r, 64), jnp.float32)], axis=1)


def _run_pre(nodes, xp, nb2, wn, wx, b1, w2, b2, ws, wd, be):
    n = nodes.shape[0]
    r = 2000
    return pl.pallas_call(
        _pre_body,
        grid=(n // r,),
        in_specs=[_rows(r, 128), _rows(r, 128), _rows(r, 1),
                  _full(wn.shape), _full(wx.shape), _full(b1.shape),
                  _full(w2.shape), _full(b2.shape), _full(ws.shape),
                  _full(wd.shape), _full(be.shape)],
        out_specs=[_rows(r, 128), _rows(r, 128), _rows(r, 128), _rows(r, NB)],
        out_shape=[jax.ShapeDtypeStruct((n, 128), jnp.float32),
                   jax.ShapeDtypeStruct((n, 128), jnp.float32),
                   jax.ShapeDtypeStruct((n, 128), jnp.float32),
                   jax.ShapeDtypeStruct((n, NB), jnp.float32)],
    )(nodes, xp, nb2, wn, wx, b1, w2, b2, ws, wd, be)


def _edge_body(megnet, gs_ref, gd_ref, edges_ref, we_ref, w2_ref, b2_ref,
               *refs):
    if megnet:
        wv_ref, virt_ref = refs[0], refs[1]
        enew_ref, eout_ref, sum_eb_ref = refs[2], refs[3], refs[4]
    else:
        enew_ref, eout_ref, sum_eb_ref, cnt_eb_ref = refs[:4]
    gs = gs_ref[...]
    oh = gs[:, 64:64 + NB]
    pre = gs[:, 0:64] + gd_ref[...][:, 0:64] + _dot(edges_ref[...], we_ref[...])
    if megnet:
        pre = pre + _dot(oh, _dot(virt_ref[...], wv_ref[...]))
    eo = _dot(_silu(pre), w2_ref[...]) + b2_ref[...]
    enew_ref[...] = edges_ref[...] + eo
    eout_ref[...] = eo

    @pl.when(pl.program_id(0) == 0)
    def _():
        sum_eb_ref[...] = jnp.zeros_like(sum_eb_ref)
        if not megnet:
            cnt_eb_ref[...] = jnp.zeros_like(cnt_eb_ref)

    sum_eb_ref[...] += _dotT(oh, eo)
    if not megnet:
        cnt_eb_ref[...] += _dotT(oh, jnp.ones_like(eo))


def _run_edge(megnet, gs, gd, edges, we, w2, b2, wv=None, virt=None):
    e = edges.shape[0]
    r = 2000
    in_specs = [_rows(r, 128), _rows(r, 128), _rows(r, 128),
                _full(we.shape), _full(w2.shape), _full(b2.shape)]
    args = [gs, gd, edges, we, w2, b2]
    if megnet:
        in_specs += [_full(wv.shape), _full(virt.shape)]
        args += [wv, virt]
    out_specs = [_rows(r, 128), _rows(r, 128), _full((NB, 128))]
    out_shape = [jax.ShapeDtypeStruct((e, 128), jnp.float32),
                 jax.ShapeDtypeStruct((e, 128), jnp.float32),
                 jax.ShapeDtypeStruct((NB, 128), jnp.float32)]
    if not megnet:
        out_specs.append(_full((NB, 128)))
        out_shape.append(jax.ShapeDtypeStruct((NB, 128), jnp.float32))
    return pl.pallas_call(
        functools.partial(_edge_body, megnet),
        grid=(e // r,),
        in_specs=in_specs,
        out_specs=out_specs,
        out_shape=out_shape,
    )(*args)


def _node_body(first, h_ref, p0_ref, p1_ref, c0_ref, c1_ref, oh_ref, wh_ref,
               wa_ref, b1_ref, w2_ref, b2_ref, *refs):
    if first:
        wsn_ref, wdn_ref, ben_ref = refs[0], refs[1], refs[2]
        hnew_ref, sumn_ref, cntn_ref, ts_ref, td_ref = refs[3:]
    else:
        wv_ref, virt_ref = refs[0], refs[1]
        hnew_ref, sumn_ref = refs[2], refs[3]
    oh = oh_ref[...]
    cnt = jnp.maximum(c0_ref[...] + c1_ref[...], 1.0)
    agg = (p0_ref[...] + p1_ref[...]) / cnt
    pre = _dot(h_ref[...], wh_ref[...]) + _dot(agg, wa_ref[...]) + b1_ref[...]
    if not first:
        pre = pre + _dot(oh, _dot(virt_ref[...], wv_ref[...]))
    nout = _dot(_silu(pre), w2_ref[...]) + b2_ref[...]
    hnew = h_ref[...] + nout
    hnew_ref[...] = hnew

    @pl.when(pl.program_id(0) == 0)
    def _():
        sumn_ref[...] = jnp.zeros_like(sumn_ref)
        if first:
            cntn_ref[...] = jnp.zeros_like(cntn_ref)

    sumn_ref[...] += _dotT(oh, nout)
    if first:
        cntn_ref[...] += _dotT(oh, jnp.ones_like(nout))
        r = oh.shape[0]
        z = jnp.zeros((r, 128 - 64 - NB), jnp.float32)
        ts_ref[...] = jnp.concatenate(
            [_dot(hnew, wsn_ref[...]) + ben_ref[...], oh, z], axis=1)
        td_ref[...] = jnp.concatenate(
            [_dot(hnew, wdn_ref[...]), jnp.zeros((r, 64), jnp.float32)],
            axis=1)


def _run_node(first, h, p0, p1, c0, c1, ohn, wh, wa, b1, w2, b2, extra):
    n = h.shape[0]
    r = 2000
    in_specs = [_rows(r, 128), _rows(r, 128), _rows(r, 128), _rows(r, 128),
                _rows(r, 128), _rows(r, NB), _full(wh.shape), _full(wa.shape),
                _full(b1.shape), _full(w2.shape), _full(b2.shape)]
    args = [h, p0, p1, c0, c1, ohn, wh, wa, b1, w2, b2]
    for a in extra:
        in_specs.append(_full(a.shape))
        args.append(a)
    out_specs = [_rows(r, 128), _full((NB, 128))]
    out_shape = [jax.ShapeDtypeStruct((n, 128), jnp.float32),
                 jax.ShapeDtypeStruct((NB, 128), jnp.float32)]
    if first:
        out_specs += [_full((NB, 128)), _rows(r, 128), _rows(r, 128)]
        out_shape += [jax.ShapeDtypeStruct((NB, 128), jnp.float32),
                      jax.ShapeDtypeStruct((n, 128), jnp.float32),
                      jax.ShapeDtypeStruct((n, 128), jnp.float32)]
    return pl.pallas_call(
        functools.partial(_node_body, first),
        grid=(n // r,),
        in_specs=in_specs,
        out_specs=out_specs,
        out_shape=out_shape,
    )(*args)


def _glob_body(sumn_ref, cntn_ref, sume_ref, cnte_ref, virt_ref, wn_ref,
               we_ref, wv_ref, b1_ref, w2_ref, b2_ref, vnew_ref):
    nmean = sumn_ref[...] / jnp.maximum(cntn_ref[...], 1.0)
    emean = sume_ref[...] / jnp.maximum(cnte_ref[...], 1.0)
    hid = _silu(_dot(nmean, wn_ref[...]) + _dot(emean, we_ref[...])
                + _dot(virt_ref[...], wv_ref[...]) + b1_ref[...])
    vnew_ref[...] = virt_ref[...] + _dot(hid, w2_ref[...]) + b2_ref[...]


def _run_glob(sumn, cntn, sume, cnte, virt, wn, we, wv, b1, w2, b2):
    return pl.pallas_call(
        _glob_body,
        out_shape=jax.ShapeDtypeStruct((NB, 128), jnp.float32),
    )(sumn, cntn, sume, cnte, virt, wn, we, wv, b1, w2, b2)


def _out_body(h_ref, pa_ref, pb_ref, pc_ref, pd_ref, pe_ref, pf_ref, c0_ref,
              c1_ref, oh_ref, virt_ref, t_ref, cond_ref, wh_ref, wa_ref,
              wuv_ref, wut_ref, wuc_ref, b1_ref, w2_ref, b2_ref, ret_ref):
    cnt = jnp.maximum(c0_ref[...] + c1_ref[...], 1.0)
    s = (pa_ref[...] + pb_ref[...] + pc_ref[...] + pd_ref[...]
         + pe_ref[...] + pf_ref[...])
    agg = s / cnt
    uproj = (_dot(virt_ref[...], wuv_ref[...])
             + _dot(cond_ref[...], wuc_ref[...])
             + _dot(t_ref[...], wut_ref[...]))
    pre = (_dot(h_ref[...], wh_ref[...]) + _dot(agg, wa_ref[...])
           + _dot(oh_ref[...], uproj) + b1_ref[...])
    ret_ref[...] = _dot(_silu(pre), w2_ref[...]) + b2_ref[...]


def _run_out(h, parts, c0, c1, ohn, virt, t, cond, wh, wa, wuv, wut, wuc, b1,
             w2, b2):
    n = h.shape[0]
    r = 2000
    dout = b2.shape[1]
    in_specs = ([_rows(r, 128)] + [_rows(r, 128)] * 6
                + [_rows(r, 128), _rows(r, 128), _rows(r, NB)]
                + [_full(a.shape) for a in
                   (virt, t, cond, wh, wa, wuv, wut, wuc, b1, w2, b2)])
    return pl.pallas_call(
        _out_body,
        grid=(n // r,),
        in_specs=in_specs,
        out_specs=_rows(r, dout),
        out_shape=jax.ShapeDtypeStruct((n, dout), jnp.float32),
    )(h, *parts, c0, c1, ohn, virt, t, cond, wh, wa, wuv, wut, wuc, b1, w2, b2)


def _mesh():
    return plsc.VectorSubcoreMesh(core_axis_name="c", subcore_axis_name="s")


def _sc_gather(src, dst, tables, idx_sel):
    e = src.shape[0]
    nt = len(tables)
    n_chunks = e // CHUNK
    per_w = n_chunks // NW
    rem = n_chunks - per_w * NW
    out_type = tuple(jax.ShapeDtypeStruct((e, t.shape[1]), jnp.float32)
                     for t in tables)
    scratch = ([pltpu.VMEM((CHUNK,), jnp.int32)] * 2
               + [pltpu.VMEM((CHUNK, t.shape[1]), jnp.float32)
                  for t in tables]
               + [pltpu.SemaphoreType.DMA])

    @functools.partial(pl.kernel, out_type=out_type, mesh=_mesh(),
                       scratch_types=scratch)
    def k(src_hbm, dst_hbm, *refs):
        tabs = refs[:nt]
        outs = refs[nt:2 * nt]
        isrc, idst = refs[2 * nt], refs[2 * nt + 1]
        bufs = refs[2 * nt + 2:3 * nt + 2]
        sem = refs[3 * nt + 2]
        cid = lax.axis_index("c")
        sid = lax.axis_index("s")
        wid = sid * NC + cid

        def do_chunk(j):
            off = pl.multiple_of(j * CHUNK, CHUNK)
            pltpu.sync_copy(src_hbm.at[pl.ds(off, CHUNK)], isrc)
            pltpu.sync_copy(dst_hbm.at[pl.ds(off, CHUNK)], idst)
            for t, sel, bf in zip(tabs, idx_sel, bufs):
                ib = isrc if sel == 0 else idst
                pltpu.async_copy(t.at[ib], bf, sem).wait()
            for bf, ot in zip(bufs, outs):
                pltpu.sync_copy(bf, ot.at[pl.ds(off, CHUNK)])

        def body(i, carry):
            do_chunk(i * NW + wid)
            return carry

        lax.fori_loop(0, per_w, body, 0)
        if rem:
            @pl.when(wid < rem)
            def _():
                do_chunk(per_w * NW + wid)

    return k(src, dst, *tables)


def _sc_scatter(v, idx, zeros_big):
    e, w = v.shape
    n = zeros_big.shape[0]
    n_chunks = e // CHUNK
    per_w = n_chunks // NW
    rem = n_chunks - per_w * NW
    rows = (n // NS) // 8 * 8
    tail = n - NS * rows
    out_type = tuple(jax.ShapeDtypeStruct((n, w), jnp.float32)
                     for _ in range(NC))
    scratch = [pltpu.VMEM((CHUNK,), jnp.int32),
               pltpu.VMEM((CHUNK, w), jnp.float32),
               pltpu.VMEM_SHARED((n, w), jnp.float32),
               pltpu.SemaphoreType.DMA]

    @functools.partial(pl.kernel, out_type=out_type, mesh=_mesh(),
                       scratch_types=scratch)
    def k(v_hbm, idx_hbm, z_hbm, out0, out1, ibuf, vbuf, acc, sem):
        cid = lax.axis_index("c")
        sid = lax.axis_index("s")
        wid = sid * NC + cid

        @pl.when(sid == 0)
        def _():
            pltpu.sync_copy(z_hbm, acc)

        plsc.subcore_barrier()

        def do_chunk(j):
            off = pl.multiple_of(j * CHUNK, CHUNK)
            pltpu.sync_copy(idx_hbm.at[pl.ds(off, CHUNK)], ibuf)
            pltpu.sync_copy(v_hbm.at[pl.ds(off, CHUNK)], vbuf)
            pltpu.sync_copy(vbuf, acc.at[ibuf], add=True)

        def body(i, carry):
            do_chunk(i * NW + wid)
            return carry

        lax.fori_loop(0, per_w, body, 0)
        if rem:
            @pl.when(wid < rem)
            def _():
                do_chunk(per_w * NW + wid)
        plsc.subcore_barrier()

        r0 = pl.multiple_of(sid * rows, 8)

        @pl.when(cid == 0)
        def _():
            pltpu.sync_copy(acc.at[pl.ds(r0, rows)], out0.at[pl.ds(r0, rows)])

            @pl.when(sid == 0)
            def _():
                if tail:
                    pltpu.sync_copy(acc.at[pl.ds(NS * rows, tail)],
                                    out0.at[pl.ds(NS * rows, tail)])

        @pl.when(cid == 1)
        def _():
            pltpu.sync_copy(acc.at[pl.ds(r0, rows)], out1.at[pl.ds(r0, rows)])

            @pl.when(sid == 0)
            def _():
                if tail:
                    pltpu.sync_copy(acc.at[pl.ds(NS * rows, tail)],
                                    out1.at[pl.ds(NS * rows, tail)])

    return k(v, idx, zeros_big)


def _sc_counts(idx, zeros_big, ones_chunk):
    e = idx.shape[0]
    n, w = zeros_big.shape
    n_chunks = e // CHUNK
    per_w = n_chunks // NW
    rem = n_chunks - per_w * NW
    rows = (n // NS) // 8 * 8
    tail = n - NS * rows
    out_type = tuple(jax.ShapeDtypeStruct((n, w), jnp.float32)
                     for _ in range(NC))
    scratch = [pltpu.VMEM((CHUNK,), jnp.int32),
               pltpu.VMEM((CHUNK, w), jnp.float32),
               pltpu.VMEM_SHARED((n, w), jnp.float32),
               pltpu.SemaphoreType.DMA]

    @functools.partial(pl.kernel, out_type=out_type, mesh=_mesh(),
                       scratch_types=scratch)
    def k(idx_hbm, z_hbm, ones_hbm, out0, out1, ibuf, obuf, acc, sem):
        cid = lax.axis_index("c")
        sid = lax.axis_index("s")
        wid = sid * NC + cid

        @pl.when(sid == 0)
        def _():
            pltpu.sync_copy(z_hbm, acc)

        pltpu.sync_copy(ones_hbm, obuf)
        plsc.subcore_barrier()

        def do_chunk(j):
            off = pl.multiple_of(j * CHUNK, CHUNK)
            pltpu.sync_copy(idx_hbm.at[pl.ds(off, CHUNK)], ibuf)
            pltpu.sync_copy(obuf, acc.at[ibuf], add=True)

        def body(i, carry):
            do_chunk(i * NW + wid)
            return carry

        lax.fori_loop(0, per_w, body, 0)
        if rem:
            @pl.when(wid < rem)
            def _():
                do_chunk(per_w * NW + wid)
        plsc.subcore_barrier()

        r0 = pl.multiple_of(sid * rows, 8)

        @pl.when(cid == 0)
        def _():
            pltpu.sync_copy(acc.at[pl.ds(r0, rows)], out0.at[pl.ds(r0, rows)])

            @pl.when(sid == 0)
            def _():
                if tail:
                    pltpu.sync_copy(acc.at[pl.ds(NS * rows, tail)],
                                    out0.at[pl.ds(NS * rows, tail)])

        @pl.when(cid == 1)
        def _():
            pltpu.sync_copy(acc.at[pl.ds(r0, rows)], out1.at[pl.ds(r0, rows)])

            @pl.when(sid == 0)
            def _():
                if tail:
                    pltpu.sync_copy(acc.at[pl.ds(NS * rows, tail)],
                                    out1.at[pl.ds(NS * rows, tail)])

    return k(idx, zeros_big, ones_chunk)


def kernel(nodes, x, edges, virtual, edge_index, node_batch, lengths, t, cond,
           params):
    n = nodes.shape[0]
    src = edge_index[0]
    dst = edge_index[1]

    (w1p, b1p), (w2p, b2p) = params["pe"]
    wn_p = w1p[:128]
    wx_p = jnp.pad(w1p[128:], ((0, 128 - (w1p.shape[0] - 128)), (0, 0)))
    xp = jnp.pad(x, ((0, 0), (0, 128 - x.shape[1])))
    nb2 = node_batch.reshape(n, 1)

    def lin(layer):
        w, b = layer
        return w, b.reshape(1, -1)

    e0w1, e0b1 = lin(params["mpnn0"]["edge"][0])
    e0w2, e0b2 = lin(params["mpnn0"]["edge"][1])
    n0w1, n0b1 = lin(params["mpnn0"]["node"][0])
    n0w2, n0b2 = lin(params["mpnn0"]["node"][1])
    g0w1, g0b1 = lin(params["mpnn0"]["glob"][0])
    g0w2, g0b2 = lin(params["mpnn0"]["glob"][1])
    e1w1, e1b1 = lin(params["mpnn1"]["edge"][0])
    e1w2, e1b2 = lin(params["mpnn1"]["edge"][1])
    n1w1, n1b1 = lin(params["mpnn1"]["node"][0])
    n1w2, n1b2 = lin(params["mpnn1"]["node"][1])
    g1w1, g1b1 = lin(params["mpnn1"]["glob"][0])
    g1w2, g1b2 = lin(params["mpnn1"]["glob"][1])
    ow1, ob1 = lin(params["out"][0])
    ow2, ob2 = lin(params["out"][1])

    zeros_big = jnp.zeros((n, 128), jnp.float32)
    ones_chunk = jnp.ones((CHUNK, 128), jnp.float32)

    h0, ts0, td0, ohn = _run_pre(
        nodes, xp, nb2, wn_p, wx_p, b1p.reshape(1, -1), w2p,
        b2p.reshape(1, -1), e0w1[0:128], e0w1[128:256], e0b1)

    c0, c1 = _sc_counts(dst, zeros_big, ones_chunk)
    si0, si1 = _sc_scatter(edges, dst, zeros_big)

    gs0, gd0 = _sc_gather(src, dst, [ts0, td0], [0, 1])
    edges1, eout0, sum_eb0, cnt_eb = _run_edge(
        False, gs0, gd0, edges, e0w1[256:384], e0w2, e0b2)
    s00, s01 = _sc_scatter(eout0, dst, zeros_big)
    h1, sum_n0, cnt_n, ts1, td1 = _run_node(
        True, h0, s00, s01, c0, c1, ohn, n0w1[0:128], n0w1[128:256], n0b1,
        n0w2, n0b2, [e1w1[0:128], e1w1[128:256], e1b1])
    virtual1 = _run_glob(sum_n0, cnt_n, sum_eb0, cnt_eb, virtual,
                         g0w1[0:128], g0w1[128:256], g0w1[256:384], g0b1,
                         g0w2, g0b2)

    gs1, gd1 = _sc_gather(src, dst, [ts1, td1], [0, 1])
    edges2, eout1, sum_eb1 = _run_edge(
        True, gs1, gd1, edges1, e1w1[256:384], e1w2, e1b2,
        wv=e1w1[384:512], virt=virtual1)
    s10, s11 = _sc_scatter(eout1, dst, zeros_big)
    h2, sum_n1 = _run_node(
        False, h1, s10, s11, c0, c1, ohn, n1w1[0:128], n1w1[128:256], n1b1,
        n1w2, n1b2, [n1w1[256:384], virtual1])
    virtual2 = _run_glob(sum_n1, cnt_n, sum_eb1, cnt_eb, virtual1,
                         g1w1[0:128], g1w1[128:256], g1w1[256:384], g1b1,
                         g1w2, g1b2)

    ret = _run_out(h2, [si0, si1, s00, s01, s10, s11], c0, c1, ohn, virtual2,
                   t, cond, ow1[0:128], ow1[128:256], ow1[256:384],
                   ow1[384:512], ow1[512:576], ob1, ow2, ob2)

    return ((h2, edges2, virtual2, edge_index, node_batch, lengths, t, cond),
            ret)


# speedup vs baseline: 6.2227x; 1.4470x over previous
import functools

import jax
import jax.numpy as jnp
from jax import lax
from jax.experimental import pallas as pl
from jax.experimental.pallas import tpu as pltpu
from jax.experimental.pallas import tpu_sc as plsc

NB = 16
NC, NS = 2, 16
NW = NC * NS
CHUNK = 128


def _silu(x):
    return x * jax.nn.sigmoid(x)


def _dot(a, b):
    return jnp.dot(a, b, preferred_element_type=jnp.float32)


def _dotT(a, b):
    return lax.dot_general(a, b, (((0,), (0,)), ((), ())),
                           preferred_element_type=jnp.float32)


def _full(shape):
    return pl.BlockSpec(shape, lambda i: (0,) * len(shape))


def _rows(r, c):
    return pl.BlockSpec((r, c), lambda i: (i, 0))


def _pre_body(nodes_ref, xp_ref, nb_ref, wn_ref, wx_ref, b1_ref, w2_ref,
              b2_ref, ws_ref, wd_ref, be_ref, h_ref, ts_ref, td_ref, oh_ref):
    r = nodes_ref.shape[0]
    hmid = _silu(_dot(nodes_ref[...], wn_ref[...])
                 + _dot(xp_ref[...], wx_ref[...]) + b1_ref[...])
    h = _dot(hmid, w2_ref[...]) + b2_ref[...]
    h_ref[...] = h
    cols = lax.broadcasted_iota(jnp.int32, (r, NB), 1)
    oh = (nb_ref[...] == cols).astype(jnp.float32)
    oh_ref[...] = oh
    z = jnp.zeros((r, 128 - 64 - NB), jnp.float32)
    ts_ref[...] = jnp.concatenate(
        [_dot(h, ws_ref[...]) + be_ref[...], oh, z], axis=1)
    td_ref[...] = jnp.concatenate(
        [_dot(h, wd_ref[...]), jnp.zeros((r, 64), jnp.float32)], axis=1)


def _run_pre(nodes, xp, nb2, wn, wx, b1, w2, b2, ws, wd, be):
    n = nodes.shape[0]
    r = 2000
    return pl.pallas_call(
        _pre_body,
        grid=(n // r,),
        in_specs=[_rows(r, 128), _rows(r, 128), _rows(r, 1),
                  _full(wn.shape), _full(wx.shape), _full(b1.shape),
                  _full(w2.shape), _full(b2.shape), _full(ws.shape),
                  _full(wd.shape), _full(be.shape)],
        out_specs=[_rows(r, 128), _rows(r, 128), _rows(r, 128), _rows(r, NB)],
        out_shape=[jax.ShapeDtypeStruct((n, 128), jnp.float32),
                   jax.ShapeDtypeStruct((n, 128), jnp.float32),
                   jax.ShapeDtypeStruct((n, 128), jnp.float32),
                   jax.ShapeDtypeStruct((n, NB), jnp.float32)],
    )(nodes, xp, nb2, wn, wx, b1, w2, b2, ws, wd, be)


def _edge_body(megnet, gs_ref, gd_ref, edges_ref, we_ref, w2_ref, b2_ref,
               *refs):
    if megnet:
        wv_ref, virt_ref = refs[0], refs[1]
        enew_ref, eout_ref, sum_eb_ref = refs[2], refs[3], refs[4]
    else:
        enew_ref, eout_ref, sum_eb_ref, cnt_eb_ref = refs[:4]
    gs = gs_ref[...]
    oh = gs[:, 64:64 + NB]
    pre = gs[:, 0:64] + gd_ref[...][:, 0:64] + _dot(edges_ref[...], we_ref[...])
    if megnet:
        pre = pre + _dot(oh, _dot(virt_ref[...], wv_ref[...]))
    eo = _dot(_silu(pre), w2_ref[...]) + b2_ref[...]
    enew_ref[...] = edges_ref[...] + eo
    eout_ref[...] = eo

    @pl.when(pl.program_id(0) == 0)
    def _():
        sum_eb_ref[...] = jnp.zeros_like(sum_eb_ref)
        if not megnet:
            cnt_eb_ref[...] = jnp.zeros_like(cnt_eb_ref)

    sum_eb_ref[...] += _dotT(oh, eo)
    if not megnet:
        cnt_eb_ref[...] += _dotT(oh, jnp.ones_like(eo))


def _run_edge(megnet, gs, gd, edges, we, w2, b2, wv=None, virt=None):
    e = edges.shape[0]
    r = 2000
    in_specs = [_rows(r, 128), _rows(r, 128), _rows(r, 128),
                _full(we.shape), _full(w2.shape), _full(b2.shape)]
    args = [gs, gd, edges, we, w2, b2]
    if megnet:
        in_specs += [_full(wv.shape), _full(virt.shape)]
        args += [wv, virt]
    out_specs = [_rows(r, 128), _rows(r, 128), _full((NB, 128))]
    out_shape = [jax.ShapeDtypeStruct((e, 128), jnp.float32),
                 jax.ShapeDtypeStruct((e, 128), jnp.float32),
                 jax.ShapeDtypeStruct((NB, 128), jnp.float32)]
    if not megnet:
        out_specs.append(_full((NB, 128)))
        out_shape.append(jax.ShapeDtypeStruct((NB, 128), jnp.float32))
    return pl.pallas_call(
        functools.partial(_edge_body, megnet),
        grid=(e // r,),
        in_specs=in_specs,
        out_specs=out_specs,
        out_shape=out_shape,
    )(*args)


def _node_body(first, h_ref, p0_ref, p1_ref, c0_ref, c1_ref, oh_ref, wh_ref,
               wa_ref, b1_ref, w2_ref, b2_ref, *refs):
    if first:
        wsn_ref, wdn_ref, ben_ref = refs[0], refs[1], refs[2]
        hnew_ref, sumn_ref, cntn_ref, ts_ref, td_ref = refs[3:]
    else:
        wv_ref, virt_ref = refs[0], refs[1]
        hnew_ref, sumn_ref = refs[2], refs[3]
    oh = oh_ref[...]
    cnt = jnp.maximum(c0_ref[...] + c1_ref[...], 1.0)
    agg = (p0_ref[...] + p1_ref[...]) / cnt
    pre = _dot(h_ref[...], wh_ref[...]) + _dot(agg, wa_ref[...]) + b1_ref[...]
    if not first:
        pre = pre + _dot(oh, _dot(virt_ref[...], wv_ref[...]))
    nout = _dot(_silu(pre), w2_ref[...]) + b2_ref[...]
    hnew = h_ref[...] + nout
    hnew_ref[...] = hnew

    @pl.when(pl.program_id(0) == 0)
    def _():
        sumn_ref[...] = jnp.zeros_like(sumn_ref)
        if first:
            cntn_ref[...] = jnp.zeros_like(cntn_ref)

    sumn_ref[...] += _dotT(oh, nout)
    if first:
        cntn_ref[...] += _dotT(oh, jnp.ones_like(nout))
        r = oh.shape[0]
        z = jnp.zeros((r, 128 - 64 - NB), jnp.float32)
        ts_ref[...] = jnp.concatenate(
            [_dot(hnew, wsn_ref[...]) + ben_ref[...], oh, z], axis=1)
        td_ref[...] = jnp.concatenate(
            [_dot(hnew, wdn_ref[...]), jnp.zeros((r, 64), jnp.float32)],
            axis=1)


def _run_node(first, h, p0, p1, c0, c1, ohn, wh, wa, b1, w2, b2, extra):
    n = h.shape[0]
    r = 2000
    in_specs = [_rows(r, 128), _rows(r, 128), _rows(r, 128), _rows(r, 128),
                _rows(r, 128), _rows(r, NB), _full(wh.shape), _full(wa.shape),
                _full(b1.shape), _full(w2.shape), _full(b2.shape)]
    args = [h, p0, p1, c0, c1, ohn, wh, wa, b1, w2, b2]
    for a in extra:
        in_specs.append(_full(a.shape))
        args.append(a)
    out_specs = [_rows(r, 128), _full((NB, 128))]
    out_shape = [jax.ShapeDtypeStruct((n, 128), jnp.float32),
                 jax.ShapeDtypeStruct((NB, 128), jnp.float32)]
    if first:
        out_specs += [_full((NB, 128)), _rows(r, 128), _rows(r, 128)]
        out_shape += [jax.ShapeDtypeStruct((NB, 128), jnp.float32),
                      jax.ShapeDtypeStruct((n, 128), jnp.float32),
                      jax.ShapeDtypeStruct((n, 128), jnp.float32)]
    return pl.pallas_call(
        functools.partial(_node_body, first),
        grid=(n // r,),
        in_specs=in_specs,
        out_specs=out_specs,
        out_shape=out_shape,
    )(*args)


def _glob_body(sumn_ref, cntn_ref, sume_ref, cnte_ref, virt_ref, wn_ref,
               we_ref, wv_ref, b1_ref, w2_ref, b2_ref, vnew_ref):
    nmean = sumn_ref[...] / jnp.maximum(cntn_ref[...], 1.0)
    emean = sume_ref[...] / jnp.maximum(cnte_ref[...], 1.0)
    hid = _silu(_dot(nmean, wn_ref[...]) + _dot(emean, we_ref[...])
                + _dot(virt_ref[...], wv_ref[...]) + b1_ref[...])
    vnew_ref[...] = virt_ref[...] + _dot(hid, w2_ref[...]) + b2_ref[...]


def _run_glob(sumn, cntn, sume, cnte, virt, wn, we, wv, b1, w2, b2):
    return pl.pallas_call(
        _glob_body,
        out_shape=jax.ShapeDtypeStruct((NB, 128), jnp.float32),
    )(sumn, cntn, sume, cnte, virt, wn, we, wv, b1, w2, b2)


def _out_body(h_ref, pa_ref, pb_ref, pc_ref, pd_ref, pe_ref, pf_ref, c0_ref,
              c1_ref, oh_ref, virt_ref, t_ref, cond_ref, wh_ref, wa_ref,
              wuv_ref, wut_ref, wuc_ref, b1_ref, w2_ref, b2_ref, ret_ref):
    cnt = jnp.maximum(c0_ref[...] + c1_ref[...], 1.0)
    s = (pa_ref[...] + pb_ref[...] + pc_ref[...] + pd_ref[...]
         + pe_ref[...] + pf_ref[...])
    agg = s / cnt
    uproj = (_dot(virt_ref[...], wuv_ref[...])
             + _dot(cond_ref[...], wuc_ref[...])
             + _dot(t_ref[...], wut_ref[...]))
    pre = (_dot(h_ref[...], wh_ref[...]) + _dot(agg, wa_ref[...])
           + _dot(oh_ref[...], uproj) + b1_ref[...])
    ret_ref[...] = _dot(_silu(pre), w2_ref[...]) + b2_ref[...]


def _run_out(h, parts, c0, c1, ohn, virt, t, cond, wh, wa, wuv, wut, wuc, b1,
             w2, b2):
    n = h.shape[0]
    r = 2000
    dout = b2.shape[1]
    in_specs = ([_rows(r, 128)] + [_rows(r, 128)] * 6
                + [_rows(r, 128), _rows(r, 128), _rows(r, NB)]
                + [_full(a.shape) for a in
                   (virt, t, cond, wh, wa, wuv, wut, wuc, b1, w2, b2)])
    return pl.pallas_call(
        _out_body,
        grid=(n // r,),
        in_specs=in_specs,
        out_specs=_rows(r, dout),
        out_shape=jax.ShapeDtypeStruct((n, dout), jnp.float32),
    )(h, *parts, c0, c1, ohn, virt, t, cond, wh, wa, wuv, wut, wuc, b1, w2, b2)


def _mesh():
    return plsc.VectorSubcoreMesh(core_axis_name="c", subcore_axis_name="s")


def _sc_gather(src, dst, tables, idx_sel):
    e = src.shape[0]
    nt = len(tables)
    n_chunks = e // CHUNK
    per_w = n_chunks // NW
    rem = n_chunks - per_w * NW
    assert per_w % 2 == 0
    out_type = tuple(jax.ShapeDtypeStruct((e, t.shape[1]), jnp.float32)
                     for t in tables)
    scratch = ([pltpu.VMEM((CHUNK,), jnp.int32)] * 4
               + [pltpu.VMEM((CHUNK, t.shape[1]), jnp.float32)
                  for t in tables for _ in range(2)]
               + [pltpu.SemaphoreType.DMA] * 6)

    @functools.partial(pl.kernel, out_type=out_type, mesh=_mesh(),
                       scratch_types=scratch)
    def k(src_hbm, dst_hbm, *refs):
        tabs = refs[:nt]
        outs = refs[nt:2 * nt]
        p = 2 * nt
        isrc = refs[p:p + 2]
        idst = refs[p + 2:p + 4]
        bufs = [refs[p + 4 + 2 * t:p + 6 + 2 * t] for t in range(nt)]
        semi = refs[p + 4 + 2 * nt:p + 6 + 2 * nt]
        semg = refs[p + 6 + 2 * nt:p + 8 + 2 * nt]
        semw = refs[p + 8 + 2 * nt:p + 10 + 2 * nt]
        cid = lax.axis_index("c")
        sid = lax.axis_index("s")
        wid = sid * NC + cid

        def off_of(i):
            return pl.multiple_of((i * NW + wid) * CHUNK, CHUNK)

        def issue_idx(i, b):
            off = off_of(i)
            pltpu.async_copy(src_hbm.at[pl.ds(off, CHUNK)], isrc[b], semi[b])
            pltpu.async_copy(dst_hbm.at[pl.ds(off, CHUNK)], idst[b], semi[b])

        def wait_idx(b):
            pltpu.make_async_copy(src_hbm.at[pl.ds(0, CHUNK)], isrc[b],
                                  semi[b]).wait()
            pltpu.make_async_copy(dst_hbm.at[pl.ds(0, CHUNK)], idst[b],
                                  semi[b]).wait()

        def run_gather(b):
            descs = []
            for t in range(nt):
                ib = isrc[b] if idx_sel[t] == 0 else idst[b]
                descs.append(pltpu.async_copy(tabs[t].at[ib], bufs[t][b],
                                              semg[b]))
            for d in descs:
                d.wait()

        def issue_wb(i, b):
            off = off_of(i)
            for t in range(nt):
                pltpu.async_copy(bufs[t][b], outs[t].at[pl.ds(off, CHUNK)],
                                 semw[b])

        def wait_wb(b):
            for t in range(nt):
                pltpu.make_async_copy(bufs[t][b],
                                      outs[t].at[pl.ds(0, CHUNK)],
                                      semw[b]).wait()

        issue_idx(0, 0)

        def body(j2, carry):
            for b in range(2):
                i = 2 * j2 + b

                @pl.when(i + 1 < per_w)
                def _():
                    issue_idx(i + 1, 1 - b)

                wait_idx(b)

                @pl.when(i >= 2)
                def _():
                    wait_wb(b)

                run_gather(b)
                issue_wb(i, b)
            return carry

        lax.fori_loop(0, per_w // 2, body, 0)
        wait_wb(0)
        wait_wb(1)
        if rem:
            @pl.when(wid < rem)
            def _():
                off = pl.multiple_of((per_w * NW + wid) * CHUNK, CHUNK)
                pltpu.sync_copy(src_hbm.at[pl.ds(off, CHUNK)], isrc[0])
                pltpu.sync_copy(dst_hbm.at[pl.ds(off, CHUNK)], idst[0])
                run_gather(0)
                for t in range(nt):
                    pltpu.sync_copy(bufs[t][0], outs[t].at[pl.ds(off, CHUNK)])

    return k(src, dst, *tables)


def _sc_scatter(v, idx, zeros_big):
    e, w = v.shape
    n = zeros_big.shape[0]
    n_chunks = e // CHUNK
    per_w = n_chunks // NW
    rem = n_chunks - per_w * NW
    rows = (n // NS) // 8 * 8
    tail = n - NS * rows
    out_type = tuple(jax.ShapeDtypeStruct((n, w), jnp.float32)
                     for _ in range(NC))
    assert per_w % 2 == 0
    scratch = [pltpu.VMEM((CHUNK,), jnp.int32)] * 2 + \
              [pltpu.VMEM((CHUNK, w), jnp.float32)] * 2 + \
              [pltpu.VMEM_SHARED((n, w), jnp.float32)] + \
              [pltpu.SemaphoreType.DMA] * 4

    @functools.partial(pl.kernel, out_type=out_type, mesh=_mesh(),
                       scratch_types=scratch)
    def k(v_hbm, idx_hbm, z_hbm, out0, out1, ibuf0, ibuf1, vbuf0, vbuf1, acc,
          seml0, seml1, sema0, sema1):
        ibuf = (ibuf0, ibuf1)
        vbuf = (vbuf0, vbuf1)
        seml = (seml0, seml1)
        sema = (sema0, sema1)
        cid = lax.axis_index("c")
        sid = lax.axis_index("s")
        wid = sid * NC + cid

        @pl.when(sid == 0)
        def _():
            pltpu.sync_copy(z_hbm, acc)

        plsc.subcore_barrier()

        def off_of(i):
            return pl.multiple_of((i * NW + wid) * CHUNK, CHUNK)

        def issue_load(i, b):
            off = off_of(i)
            pltpu.async_copy(idx_hbm.at[pl.ds(off, CHUNK)], ibuf[b], seml[b])
            pltpu.async_copy(v_hbm.at[pl.ds(off, CHUNK)], vbuf[b], seml[b])

        def wait_load(b):
            pltpu.make_async_copy(idx_hbm.at[pl.ds(0, CHUNK)], ibuf[b],
                                  seml[b]).wait()
            pltpu.make_async_copy(v_hbm.at[pl.ds(0, CHUNK)], vbuf[b],
                                  seml[b]).wait()

        issue_load(0, 0)

        def body(j2, carry):
            for b in range(2):
                i = 2 * j2 + b

                @pl.when(i + 1 < per_w)
                def _():
                    issue_load(i + 1, 1 - b)

                wait_load(b)
                pltpu.sync_copy(vbuf[b], acc.at[ibuf[b]], add=True)
            return carry

        lax.fori_loop(0, per_w // 2, body, 0)
        if rem:
            @pl.when(wid < rem)
            def _():
                off = pl.multiple_of((per_w * NW + wid) * CHUNK, CHUNK)
                pltpu.sync_copy(idx_hbm.at[pl.ds(off, CHUNK)], ibuf[0])
                pltpu.sync_copy(v_hbm.at[pl.ds(off, CHUNK)], vbuf[0])
                pltpu.sync_copy(vbuf[0], acc.at[ibuf[0]], add=True)
        plsc.subcore_barrier()

        r0 = pl.multiple_of(sid * rows, 8)

        @pl.when(cid == 0)
        def _():
            pltpu.sync_copy(acc.at[pl.ds(r0, rows)], out0.at[pl.ds(r0, rows)])

            @pl.when(sid == 0)
            def _():
                if tail:
                    pltpu.sync_copy(acc.at[pl.ds(NS * rows, tail)],
                                    out0.at[pl.ds(NS * rows, tail)])

        @pl.when(cid == 1)
        def _():
            pltpu.sync_copy(acc.at[pl.ds(r0, rows)], out1.at[pl.ds(r0, rows)])

            @pl.when(sid == 0)
            def _():
                if tail:
                    pltpu.sync_copy(acc.at[pl.ds(NS * rows, tail)],
                                    out1.at[pl.ds(NS * rows, tail)])

    return k(v, idx, zeros_big)


def _sc_counts(idx, zeros_big, ones_chunk):
    e = idx.shape[0]
    n, w = zeros_big.shape
    n_chunks = e // CHUNK
    per_w = n_chunks // NW
    rem = n_chunks - per_w * NW
    rows = (n // NS) // 8 * 8
    tail = n - NS * rows
    out_type = tuple(jax.ShapeDtypeStruct((n, w), jnp.float32)
                     for _ in range(NC))
    assert per_w % 2 == 0
    scratch = [pltpu.VMEM((CHUNK,), jnp.int32)] * 2 + \
              [pltpu.VMEM((CHUNK, w), jnp.float32)] + \
              [pltpu.VMEM_SHARED((n, w), jnp.float32)] + \
              [pltpu.SemaphoreType.DMA] * 4

    @functools.partial(pl.kernel, out_type=out_type, mesh=_mesh(),
                       scratch_types=scratch)
    def k(idx_hbm, z_hbm, ones_hbm, out0, out1, ibuf0, ibuf1, obuf, acc,
          seml0, seml1, sema0, sema1):
        ibuf = (ibuf0, ibuf1)
        seml = (seml0, seml1)
        sema = (sema0, sema1)
        cid = lax.axis_index("c")
        sid = lax.axis_index("s")
        wid = sid * NC + cid

        @pl.when(sid == 0)
        def _():
            pltpu.sync_copy(z_hbm, acc)

        pltpu.sync_copy(ones_hbm, obuf)
        plsc.subcore_barrier()

        def off_of(i):
            return pl.multiple_of((i * NW + wid) * CHUNK, CHUNK)

        def issue_load(i, b):
            pltpu.async_copy(idx_hbm.at[pl.ds(off_of(i), CHUNK)], ibuf[b],
                             seml[b])

        def wait_load(b):
            pltpu.make_async_copy(idx_hbm.at[pl.ds(0, CHUNK)], ibuf[b],
                                  seml[b]).wait()

        issue_load(0, 0)

        def body(j2, carry):
            for b in range(2):
                i = 2 * j2 + b

                @pl.when(i + 1 < per_w)
                def _():
                    issue_load(i + 1, 1 - b)

                wait_load(b)
                pltpu.sync_copy(obuf, acc.at[ibuf[b]], add=True)
            return carry

        lax.fori_loop(0, per_w // 2, body, 0)
        if rem:
            @pl.when(wid < rem)
            def _():
                off = pl.multiple_of((per_w * NW + wid) * CHUNK, CHUNK)
                pltpu.sync_copy(idx_hbm.at[pl.ds(off, CHUNK)], ibuf[0])
                pltpu.sync_copy(obuf, acc.at[ibuf[0]], add=True)
        plsc.subcore_barrier()

        r0 = pl.multiple_of(sid * rows, 8)

        @pl.when(cid == 0)
        def _():
            pltpu.sync_copy(acc.at[pl.ds(r0, rows)], out0.at[pl.ds(r0, rows)])

            @pl.when(sid == 0)
            def _():
                if tail:
                    pltpu.sync_copy(acc.at[pl.ds(NS * rows, tail)],
                                    out0.at[pl.ds(NS * rows, tail)])

        @pl.when(cid == 1)
        def _():
            pltpu.sync_copy(acc.at[pl.ds(r0, rows)], out1.at[pl.ds(r0, rows)])

            @pl.when(sid == 0)
            def _():
                if tail:
                    pltpu.sync_copy(acc.at[pl.ds(NS * rows, tail)],
                                    out1.at[pl.ds(NS * rows, tail)])

    return k(idx, zeros_big, ones_chunk)


def kernel(nodes, x, edges, virtual, edge_index, node_batch, lengths, t, cond,
           params):
    n = nodes.shape[0]
    src = edge_index[0]
    dst = edge_index[1]

    (w1p, b1p), (w2p, b2p) = params["pe"]
    wn_p = w1p[:128]
    wx_p = jnp.pad(w1p[128:], ((0, 128 - (w1p.shape[0] - 128)), (0, 0)))
    xp = jnp.pad(x, ((0, 0), (0, 128 - x.shape[1])))
    nb2 = node_batch.reshape(n, 1)

    def lin(layer):
        w, b = layer
        return w, b.reshape(1, -1)

    e0w1, e0b1 = lin(params["mpnn0"]["edge"][0])
    e0w2, e0b2 = lin(params["mpnn0"]["edge"][1])
    n0w1, n0b1 = lin(params["mpnn0"]["node"][0])
    n0w2, n0b2 = lin(params["mpnn0"]["node"][1])
    g0w1, g0b1 = lin(params["mpnn0"]["glob"][0])
    g0w2, g0b2 = lin(params["mpnn0"]["glob"][1])
    e1w1, e1b1 = lin(params["mpnn1"]["edge"][0])
    e1w2, e1b2 = lin(params["mpnn1"]["edge"][1])
    n1w1, n1b1 = lin(params["mpnn1"]["node"][0])
    n1w2, n1b2 = lin(params["mpnn1"]["node"][1])
    g1w1, g1b1 = lin(params["mpnn1"]["glob"][0])
    g1w2, g1b2 = lin(params["mpnn1"]["glob"][1])
    ow1, ob1 = lin(params["out"][0])
    ow2, ob2 = lin(params["out"][1])

    zeros_big = jnp.zeros((n, 128), jnp.float32)
    ones_chunk = jnp.ones((CHUNK, 128), jnp.float32)

    h0, ts0, td0, ohn = _run_pre(
        nodes, xp, nb2, wn_p, wx_p, b1p.reshape(1, -1), w2p,
        b2p.reshape(1, -1), e0w1[0:128], e0w1[128:256], e0b1)

    c0, c1 = _sc_counts(dst, zeros_big, ones_chunk)
    si0, si1 = _sc_scatter(edges, dst, zeros_big)

    gs0, gd0 = _sc_gather(src, dst, [ts0, td0], [0, 1])
    edges1, eout0, sum_eb0, cnt_eb = _run_edge(
        False, gs0, gd0, edges, e0w1[256:384], e0w2, e0b2)
    s00, s01 = _sc_scatter(eout0, dst, zeros_big)
    h1, sum_n0, cnt_n, ts1, td1 = _run_node(
        True, h0, s00, s01, c0, c1, ohn, n0w1[0:128], n0w1[128:256], n0b1,
        n0w2, n0b2, [e1w1[0:128], e1w1[128:256], e1b1])
    virtual1 = _run_glob(sum_n0, cnt_n, sum_eb0, cnt_eb, virtual,
                         g0w1[0:128], g0w1[128:256], g0w1[256:384], g0b1,
                         g0w2, g0b2)

    gs1, gd1 = _sc_gather(src, dst, [ts1, td1], [0, 1])
    edges2, eout1, sum_eb1 = _run_edge(
        True, gs1, gd1, edges1, e1w1[256:384], e1w2, e1b2,
        wv=e1w1[384:512], virt=virtual1)
    s10, s11 = _sc_scatter(eout1, dst, zeros_big)
    h2, sum_n1 = _run_node(
        False, h1, s10, s11, c0, c1, ohn, n1w1[0:128], n1w1[128:256], n1b1,
        n1w2, n1b2, [n1w1[256:384], virtual1])
    virtual2 = _run_glob(sum_n1, cnt_n, sum_eb1, cnt_eb, virtual1,
                         g1w1[0:128], g1w1[128:256], g1w1[256:384], g1b1,
                         g1w2, g1b2)

    ret = _run_out(h2, [si0, si1, s00, s01, s10, s11], c0, c1, ohn, virtual2,
                   t, cond, ow1[0:128], ow1[128:256], ow1[256:384],
                   ow1[384:512], ow1[512:576], ob1, ow2, ob2)

    return ((h2, edges2, virtual2, edge_index, node_batch, lengths, t, cond),
            ret)


# speedup vs baseline: 6.7106x; 1.0784x over previous
import functools

import jax
import jax.numpy as jnp
from jax import lax
from jax.experimental import pallas as pl
from jax.experimental.pallas import tpu as pltpu
from jax.experimental.pallas import tpu_sc as plsc

NB = 16
NC, NS = 2, 16
NW = NC * NS
CHUNK = 128


def _silu(x):
    return x * jax.nn.sigmoid(x)


def _dot(a, b):
    return jnp.dot(a, b, preferred_element_type=jnp.float32)


def _dotT(a, b):
    return lax.dot_general(a, b, (((0,), (0,)), ((), ())),
                           preferred_element_type=jnp.float32)


def _full(shape):
    return pl.BlockSpec(shape, lambda i: (0,) * len(shape))


def _rows(r, c):
    return pl.BlockSpec((r, c), lambda i: (i, 0))


def _pre_body(nodes_ref, xp_ref, nb_ref, wn_ref, wx_ref, b1_ref, w2_ref,
              b2_ref, ws_ref, wd_ref, be_ref, h_ref, ts_ref, td_ref, oh_ref):
    r = nodes_ref.shape[0]
    hmid = _silu(_dot(nodes_ref[...], wn_ref[...])
                 + _dot(xp_ref[...], wx_ref[...]) + b1_ref[...])
    h = _dot(hmid, w2_ref[...]) + b2_ref[...]
    h_ref[...] = h
    cols = lax.broadcasted_iota(jnp.int32, (r, NB), 1)
    oh = (nb_ref[...] == cols).astype(jnp.float32)
    oh_ref[...] = oh
    z = jnp.zeros((r, 128 - 64 - NB), jnp.float32)
    ts_ref[...] = jnp.concatenate(
        [_dot(h, ws_ref[...]) + be_ref[...], oh, z], axis=1)
    td_ref[...] = jnp.concatenate(
        [_dot(h, wd_ref[...]), jnp.zeros((r, 64), jnp.float32)], axis=1)


def _run_pre(nodes, xp, nb2, wn, wx, b1, w2, b2, ws, wd, be):
    n = nodes.shape[0]
    r = 2000
    return pl.pallas_call(
        _pre_body,
        grid=(n // r,),
        in_specs=[_rows(r, 128), _rows(r, 128), _rows(r, 1),
                  _full(wn.shape), _full(wx.shape), _full(b1.shape),
                  _full(w2.shape), _full(b2.shape), _full(ws.shape),
                  _full(wd.shape), _full(be.shape)],
        out_specs=[_rows(r, 128), _rows(r, 128), _rows(r, 128), _rows(r, NB)],
        out_shape=[jax.ShapeDtypeStruct((n, 128), jnp.float32),
                   jax.ShapeDtypeStruct((n, 128), jnp.float32),
                   jax.ShapeDtypeStruct((n, 128), jnp.float32),
                   jax.ShapeDtypeStruct((n, NB), jnp.float32)],
    )(nodes, xp, nb2, wn, wx, b1, w2, b2, ws, wd, be)


def _edge_body(megnet, gs_ref, gd_ref, edges_ref, we_ref, w2_ref, b2_ref,
               *refs):
    if megnet:
        wv_ref, virt_ref = refs[0], refs[1]
        enew_ref, sum_eb_ref = refs[2], refs[3]
    else:
        enew_ref, sum_eb_ref, cnt_eb_ref = refs[:3]
    gs = gs_ref[...]
    oh = gs[:, 64:64 + NB]
    pre = gs[:, 0:64] + gd_ref[...][:, 0:64] + _dot(edges_ref[...], we_ref[...])
    if megnet:
        pre = pre + _dot(oh, _dot(virt_ref[...], wv_ref[...]))
    eo = _dot(_silu(pre), w2_ref[...]) + b2_ref[...]
    enew_ref[...] = edges_ref[...] + eo

    @pl.when(pl.program_id(0) == 0)
    def _():
        sum_eb_ref[...] = jnp.zeros_like(sum_eb_ref)
        if not megnet:
            cnt_eb_ref[...] = jnp.zeros_like(cnt_eb_ref)

    sum_eb_ref[...] += _dotT(oh, eo)
    if not megnet:
        cnt_eb_ref[...] += _dotT(oh, jnp.ones_like(eo))


def _run_edge(megnet, gs, gd, edges, we, w2, b2, wv=None, virt=None):
    e = edges.shape[0]
    r = 2000
    in_specs = [_rows(r, 128), _rows(r, 128), _rows(r, 128),
                _full(we.shape), _full(w2.shape), _full(b2.shape)]
    args = [gs, gd, edges, we, w2, b2]
    if megnet:
        in_specs += [_full(wv.shape), _full(virt.shape)]
        args += [wv, virt]
    out_specs = [_rows(r, 128), _full((NB, 128))]
    out_shape = [jax.ShapeDtypeStruct((e, 128), jnp.float32),
                 jax.ShapeDtypeStruct((NB, 128), jnp.float32)]
    if not megnet:
        out_specs.append(_full((NB, 128)))
        out_shape.append(jax.ShapeDtypeStruct((NB, 128), jnp.float32))
    return pl.pallas_call(
        functools.partial(_edge_body, megnet),
        grid=(e // r,),
        in_specs=in_specs,
        out_specs=out_specs,
        out_shape=out_shape,
    )(*args)


def _node_body(first, h_ref, pa0_ref, pa1_ref, pb0_ref, pb1_ref, c0_ref,
               c1_ref, oh_ref, wh_ref, wa_ref, b1_ref, w2_ref, b2_ref, *refs):
    if first:
        wsn_ref, wdn_ref, ben_ref = refs[0], refs[1], refs[2]
        hnew_ref, sumn_ref, cntn_ref, ts_ref, td_ref = refs[3:]
    else:
        wv_ref, virt_ref = refs[0], refs[1]
        hnew_ref, sumn_ref = refs[2], refs[3]
    oh = oh_ref[...]
    cnt = jnp.maximum(c0_ref[...] + c1_ref[...], 1.0)
    agg = ((pa0_ref[...] + pa1_ref[...])
           - (pb0_ref[...] + pb1_ref[...])) / cnt
    pre = _dot(h_ref[...], wh_ref[...]) + _dot(agg, wa_ref[...]) + b1_ref[...]
    if not first:
        pre = pre + _dot(oh, _dot(virt_ref[...], wv_ref[...]))
    nout = _dot(_silu(pre), w2_ref[...]) + b2_ref[...]
    hnew = h_ref[...] + nout
    hnew_ref[...] = hnew

    @pl.when(pl.program_id(0) == 0)
    def _():
        sumn_ref[...] = jnp.zeros_like(sumn_ref)
        if first:
            cntn_ref[...] = jnp.zeros_like(cntn_ref)

    sumn_ref[...] += _dotT(oh, nout)
    if first:
        cntn_ref[...] += _dotT(oh, jnp.ones_like(nout))
        r = oh.shape[0]
        z = jnp.zeros((r, 128 - 64 - NB), jnp.float32)
        ts_ref[...] = jnp.concatenate(
            [_dot(hnew, wsn_ref[...]) + ben_ref[...], oh, z], axis=1)
        td_ref[...] = jnp.concatenate(
            [_dot(hnew, wdn_ref[...]), jnp.zeros((r, 64), jnp.float32)],
            axis=1)


def _run_node(first, h, pa0, pa1, pb0, pb1, c0, c1, ohn, wh, wa, b1, w2, b2,
              extra):
    n = h.shape[0]
    r = 2000
    in_specs = [_rows(r, 128)] * 7 + [_rows(r, NB), _full(wh.shape),
                                      _full(wa.shape), _full(b1.shape),
                                      _full(w2.shape), _full(b2.shape)]
    args = [h, pa0, pa1, pb0, pb1, c0, c1, ohn, wh, wa, b1, w2, b2]
    for a in extra:
        in_specs.append(_full(a.shape))
        args.append(a)
    out_specs = [_rows(r, 128), _full((NB, 128))]
    out_shape = [jax.ShapeDtypeStruct((n, 128), jnp.float32),
                 jax.ShapeDtypeStruct((NB, 128), jnp.float32)]
    if first:
        out_specs += [_full((NB, 128)), _rows(r, 128), _rows(r, 128)]
        out_shape += [jax.ShapeDtypeStruct((NB, 128), jnp.float32),
                      jax.ShapeDtypeStruct((n, 128), jnp.float32),
                      jax.ShapeDtypeStruct((n, 128), jnp.float32)]
    return pl.pallas_call(
        functools.partial(_node_body, first),
        grid=(n // r,),
        in_specs=in_specs,
        out_specs=out_specs,
        out_shape=out_shape,
    )(*args)


def _glob_body(sumn_ref, cntn_ref, sume_ref, cnte_ref, virt_ref, wn_ref,
               we_ref, wv_ref, b1_ref, w2_ref, b2_ref, vnew_ref):
    nmean = sumn_ref[...] / jnp.maximum(cntn_ref[...], 1.0)
    emean = sume_ref[...] / jnp.maximum(cnte_ref[...], 1.0)
    hid = _silu(_dot(nmean, wn_ref[...]) + _dot(emean, we_ref[...])
                + _dot(virt_ref[...], wv_ref[...]) + b1_ref[...])
    vnew_ref[...] = virt_ref[...] + _dot(hid, w2_ref[...]) + b2_ref[...]


def _run_glob(sumn, cntn, sume, cnte, virt, wn, we, wv, b1, w2, b2):
    return pl.pallas_call(
        _glob_body,
        out_shape=jax.ShapeDtypeStruct((NB, 128), jnp.float32),
    )(sumn, cntn, sume, cnte, virt, wn, we, wv, b1, w2, b2)


def _out_body(h_ref, pa_ref, pb_ref, c0_ref, c1_ref, oh_ref, virt_ref, t_ref,
              cond_ref, wh_ref, wa_ref, wuv_ref, wut_ref, wuc_ref, b1_ref,
              w2_ref, b2_ref, ret_ref):
    cnt = jnp.maximum(c0_ref[...] + c1_ref[...], 1.0)
    agg = (pa_ref[...] + pb_ref[...]) / cnt
    uproj = (_dot(virt_ref[...], wuv_ref[...])
             + _dot(cond_ref[...], wuc_ref[...])
             + _dot(t_ref[...], wut_ref[...]))
    pre = (_dot(h_ref[...], wh_ref[...]) + _dot(agg, wa_ref[...])
           + _dot(oh_ref[...], uproj) + b1_ref[...])
    ret_ref[...] = _dot(_silu(pre), w2_ref[...]) + b2_ref[...]


def _run_out(h, parts, c0, c1, ohn, virt, t, cond, wh, wa, wuv, wut, wuc, b1,
             w2, b2):
    n = h.shape[0]
    r = 2000
    dout = b2.shape[1]
    in_specs = ([_rows(r, 128)] + [_rows(r, 128)] * 2
                + [_rows(r, 128), _rows(r, 128), _rows(r, NB)]
                + [_full(a.shape) for a in
                   (virt, t, cond, wh, wa, wuv, wut, wuc, b1, w2, b2)])
    return pl.pallas_call(
        _out_body,
        grid=(n // r,),
        in_specs=in_specs,
        out_specs=_rows(r, dout),
        out_shape=jax.ShapeDtypeStruct((n, dout), jnp.float32),
    )(h, *parts, c0, c1, ohn, virt, t, cond, wh, wa, wuv, wut, wuc, b1, w2, b2)


def _mesh():
    return plsc.VectorSubcoreMesh(core_axis_name="c", subcore_axis_name="s")


def _sc_gather(src, dst, tables, idx_sel):
    e = src.shape[0]
    nt = len(tables)
    n_chunks = e // CHUNK
    per_w = n_chunks // NW
    rem = n_chunks - per_w * NW
    assert per_w % 2 == 0
    out_type = tuple(jax.ShapeDtypeStruct((e, t.shape[1]), jnp.float32)
                     for t in tables)
    scratch = ([pltpu.VMEM((CHUNK,), jnp.int32)] * 4
               + [pltpu.VMEM((CHUNK, t.shape[1]), jnp.float32)
                  for t in tables for _ in range(2)]
               + [pltpu.SemaphoreType.DMA] * 6)

    @functools.partial(pl.kernel, out_type=out_type, mesh=_mesh(),
                       scratch_types=scratch)
    def k(src_hbm, dst_hbm, *refs):
        tabs = refs[:nt]
        outs = refs[nt:2 * nt]
        p = 2 * nt
        isrc = refs[p:p + 2]
        idst = refs[p + 2:p + 4]
        bufs = [refs[p + 4 + 2 * t:p + 6 + 2 * t] for t in range(nt)]
        semi = refs[p + 4 + 2 * nt:p + 6 + 2 * nt]
        semg = refs[p + 6 + 2 * nt:p + 8 + 2 * nt]
        semw = refs[p + 8 + 2 * nt:p + 10 + 2 * nt]
        cid = lax.axis_index("c")
        sid = lax.axis_index("s")
        wid = sid * NC + cid

        def off_of(i):
            return pl.multiple_of((i * NW + wid) * CHUNK, CHUNK)

        def issue_idx(i, b):
            off = off_of(i)
            pltpu.async_copy(src_hbm.at[pl.ds(off, CHUNK)], isrc[b], semi[b])
            pltpu.async_copy(dst_hbm.at[pl.ds(off, CHUNK)], idst[b], semi[b])

        def wait_idx(b):
            pltpu.make_async_copy(src_hbm.at[pl.ds(0, CHUNK)], isrc[b],
                                  semi[b]).wait()
            pltpu.make_async_copy(dst_hbm.at[pl.ds(0, CHUNK)], idst[b],
                                  semi[b]).wait()

        def run_gather(b):
            descs = []
            for t in range(nt):
                ib = isrc[b] if idx_sel[t] == 0 else idst[b]
                descs.append(pltpu.async_copy(tabs[t].at[ib], bufs[t][b],
                                              semg[b]))
            for d in descs:
                d.wait()

        def issue_wb(i, b):
            off = off_of(i)
            for t in range(nt):
                pltpu.async_copy(bufs[t][b], outs[t].at[pl.ds(off, CHUNK)],
                                 semw[b])

        def wait_wb(b):
            for t in range(nt):
                pltpu.make_async_copy(bufs[t][b],
                                      outs[t].at[pl.ds(0, CHUNK)],
                                      semw[b]).wait()

        issue_idx(0, 0)

        def body(j2, carry):
            for b in range(2):
                i = 2 * j2 + b

                @pl.when(i + 1 < per_w)
                def _():
                    issue_idx(i + 1, 1 - b)

                wait_idx(b)

                @pl.when(i >= 2)
                def _():
                    wait_wb(b)

                run_gather(b)
                issue_wb(i, b)
            return carry

        lax.fori_loop(0, per_w // 2, body, 0)
        wait_wb(0)
        wait_wb(1)
        if rem:
            @pl.when(wid < rem)
            def _():
                off = pl.multiple_of((per_w * NW + wid) * CHUNK, CHUNK)
                pltpu.sync_copy(src_hbm.at[pl.ds(off, CHUNK)], isrc[0])
                pltpu.sync_copy(dst_hbm.at[pl.ds(off, CHUNK)], idst[0])
                run_gather(0)
                for t in range(nt):
                    pltpu.sync_copy(bufs[t][0], outs[t].at[pl.ds(off, CHUNK)])

    return k(src, dst, *tables)


def _sc_scatter(v, idx, zeros_big):
    e, w = v.shape
    n = zeros_big.shape[0]
    n_chunks = e // CHUNK
    per_w = n_chunks // NW
    rem = n_chunks - per_w * NW
    rows = (n // NS) // 8 * 8
    tail = n - NS * rows
    out_type = tuple(jax.ShapeDtypeStruct((n, w), jnp.float32)
                     for _ in range(NC))
    assert per_w % 2 == 0
    scratch = [pltpu.VMEM((CHUNK,), jnp.int32)] * 2 + \
              [pltpu.VMEM((CHUNK, w), jnp.float32)] * 2 + \
              [pltpu.VMEM_SHARED((n, w), jnp.float32)] + \
              [pltpu.SemaphoreType.DMA] * 4

    @functools.partial(pl.kernel, out_type=out_type, mesh=_mesh(),
                       scratch_types=scratch)
    def k(v_hbm, idx_hbm, z_hbm, out0, out1, ibuf0, ibuf1, vbuf0, vbuf1, acc,
          seml0, seml1, sema0, sema1):
        ibuf = (ibuf0, ibuf1)
        vbuf = (vbuf0, vbuf1)
        seml = (seml0, seml1)
        sema = (sema0, sema1)
        cid = lax.axis_index("c")
        sid = lax.axis_index("s")
        wid = sid * NC + cid

        @pl.when(sid == 0)
        def _():
            pltpu.sync_copy(z_hbm, acc)

        plsc.subcore_barrier()

        def off_of(i):
            return pl.multiple_of((i * NW + wid) * CHUNK, CHUNK)

        def issue_load(i, b):
            off = off_of(i)
            pltpu.async_copy(idx_hbm.at[pl.ds(off, CHUNK)], ibuf[b], seml[b])
            pltpu.async_copy(v_hbm.at[pl.ds(off, CHUNK)], vbuf[b], seml[b])

        def wait_load(b):
            pltpu.make_async_copy(idx_hbm.at[pl.ds(0, CHUNK)], ibuf[b],
                                  seml[b]).wait()
            pltpu.make_async_copy(v_hbm.at[pl.ds(0, CHUNK)], vbuf[b],
                                  seml[b]).wait()

        issue_load(0, 0)

        def body(j2, carry):
            for b in range(2):
                i = 2 * j2 + b

                @pl.when(i + 1 < per_w)
                def _():
                    issue_load(i + 1, 1 - b)

                wait_load(b)
                pltpu.sync_copy(vbuf[b], acc.at[ibuf[b]], add=True)
            return carry

        lax.fori_loop(0, per_w // 2, body, 0)
        if rem:
            @pl.when(wid < rem)
            def _():
                off = pl.multiple_of((per_w * NW + wid) * CHUNK, CHUNK)
                pltpu.sync_copy(idx_hbm.at[pl.ds(off, CHUNK)], ibuf[0])
                pltpu.sync_copy(v_hbm.at[pl.ds(off, CHUNK)], vbuf[0])
                pltpu.sync_copy(vbuf[0], acc.at[ibuf[0]], add=True)
        plsc.subcore_barrier()

        r0 = pl.multiple_of(sid * rows, 8)

        @pl.when(cid == 0)
        def _():
            pltpu.sync_copy(acc.at[pl.ds(r0, rows)], out0.at[pl.ds(r0, rows)])

            @pl.when(sid == 0)
            def _():
                if tail:
                    pltpu.sync_copy(acc.at[pl.ds(NS * rows, tail)],
                                    out0.at[pl.ds(NS * rows, tail)])

        @pl.when(cid == 1)
        def _():
            pltpu.sync_copy(acc.at[pl.ds(r0, rows)], out1.at[pl.ds(r0, rows)])

            @pl.when(sid == 0)
            def _():
                if tail:
                    pltpu.sync_copy(acc.at[pl.ds(NS * rows, tail)],
                                    out1.at[pl.ds(NS * rows, tail)])

    return k(v, idx, zeros_big)


def _sc_counts(idx, zeros_big, ones_chunk):
    e = idx.shape[0]
    n, w = zeros_big.shape
    n_chunks = e // CHUNK
    per_w = n_chunks // NW
    rem = n_chunks - per_w * NW
    rows = (n // NS) // 8 * 8
    tail = n - NS * rows
    out_type = tuple(jax.ShapeDtypeStruct((n, w), jnp.float32)
                     for _ in range(NC))
    assert per_w % 2 == 0
    scratch = [pltpu.VMEM((CHUNK,), jnp.int32)] * 2 + \
              [pltpu.VMEM((CHUNK, w), jnp.float32)] + \
              [pltpu.VMEM_SHARED((n, w), jnp.float32)] + \
              [pltpu.SemaphoreType.DMA] * 4

    @functools.partial(pl.kernel, out_type=out_type, mesh=_mesh(),
                       scratch_types=scratch)
    def k(idx_hbm, z_hbm, ones_hbm, out0, out1, ibuf0, ibuf1, obuf, acc,
          seml0, seml1, sema0, sema1):
        ibuf = (ibuf0, ibuf1)
        seml = (seml0, seml1)
        sema = (sema0, sema1)
        cid = lax.axis_index("c")
        sid = lax.axis_index("s")
        wid = sid * NC + cid

        @pl.when(sid == 0)
        def _():
            pltpu.sync_copy(z_hbm, acc)

        pltpu.sync_copy(ones_hbm, obuf)
        plsc.subcore_barrier()

        def off_of(i):
            return pl.multiple_of((i * NW + wid) * CHUNK, CHUNK)

        def issue_load(i, b):
            pltpu.async_copy(idx_hbm.at[pl.ds(off_of(i), CHUNK)], ibuf[b],
                             seml[b])

        def wait_load(b):
            pltpu.make_async_copy(idx_hbm.at[pl.ds(0, CHUNK)], ibuf[b],
                                  seml[b]).wait()

        issue_load(0, 0)

        def body(j2, carry):
            for b in range(2):
                i = 2 * j2 + b

                @pl.when(i + 1 < per_w)
                def _():
                    issue_load(i + 1, 1 - b)

                wait_load(b)
                pltpu.sync_copy(obuf, acc.at[ibuf[b]], add=True)
            return carry

        lax.fori_loop(0, per_w // 2, body, 0)
        if rem:
            @pl.when(wid < rem)
            def _():
                off = pl.multiple_of((per_w * NW + wid) * CHUNK, CHUNK)
                pltpu.sync_copy(idx_hbm.at[pl.ds(off, CHUNK)], ibuf[0])
                pltpu.sync_copy(obuf, acc.at[ibuf[0]], add=True)
        plsc.subcore_barrier()

        r0 = pl.multiple_of(sid * rows, 8)

        @pl.when(cid == 0)
        def _():
            pltpu.sync_copy(acc.at[pl.ds(r0, rows)], out0.at[pl.ds(r0, rows)])

            @pl.when(sid == 0)
            def _():
                if tail:
                    pltpu.sync_copy(acc.at[pl.ds(NS * rows, tail)],
                                    out0.at[pl.ds(NS * rows, tail)])

        @pl.when(cid == 1)
        def _():
            pltpu.sync_copy(acc.at[pl.ds(r0, rows)], out1.at[pl.ds(r0, rows)])

            @pl.when(sid == 0)
            def _():
                if tail:
                    pltpu.sync_copy(acc.at[pl.ds(NS * rows, tail)],
                                    out1.at[pl.ds(NS * rows, tail)])

    return k(idx, zeros_big, ones_chunk)


def kernel(nodes, x, edges, virtual, edge_index, node_batch, lengths, t, cond,
           params):
    n = nodes.shape[0]
    src = edge_index[0]
    dst = edge_index[1]

    (w1p, b1p), (w2p, b2p) = params["pe"]
    wn_p = w1p[:128]
    wx_p = jnp.pad(w1p[128:], ((0, 128 - (w1p.shape[0] - 128)), (0, 0)))
    xp = jnp.pad(x, ((0, 0), (0, 128 - x.shape[1])))
    nb2 = node_batch.reshape(n, 1)

    def lin(layer):
        w, b = layer
        return w, b.reshape(1, -1)

    e0w1, e0b1 = lin(params["mpnn0"]["edge"][0])
    e0w2, e0b2 = lin(params["mpnn0"]["edge"][1])
    n0w1, n0b1 = lin(params["mpnn0"]["node"][0])
    n0w2, n0b2 = lin(params["mpnn0"]["node"][1])
    g0w1, g0b1 = lin(params["mpnn0"]["glob"][0])
    g0w2, g0b2 = lin(params["mpnn0"]["glob"][1])
    e1w1, e1b1 = lin(params["mpnn1"]["edge"][0])
    e1w2, e1b2 = lin(params["mpnn1"]["edge"][1])
    n1w1, n1b1 = lin(params["mpnn1"]["node"][0])
    n1w2, n1b2 = lin(params["mpnn1"]["node"][1])
    g1w1, g1b1 = lin(params["mpnn1"]["glob"][0])
    g1w2, g1b2 = lin(params["mpnn1"]["glob"][1])
    ow1, ob1 = lin(params["out"][0])
    ow2, ob2 = lin(params["out"][1])

    zeros_big = jnp.zeros((n, 128), jnp.float32)
    ones_chunk = jnp.ones((CHUNK, 128), jnp.float32)

    h0, ts0, td0, ohn = _run_pre(
        nodes, xp, nb2, wn_p, wx_p, b1p.reshape(1, -1), w2p,
        b2p.reshape(1, -1), e0w1[0:128], e0w1[128:256], e0b1)

    c0, c1 = _sc_counts(dst, zeros_big, ones_chunk)
    si0, si1 = _sc_scatter(edges, dst, zeros_big)

    gs0, gd0 = _sc_gather(src, dst, [ts0, td0], [0, 1])
    edges1, sum_eb0, cnt_eb = _run_edge(
        False, gs0, gd0, edges, e0w1[256:384], e0w2, e0b2)
    t10, t11 = _sc_scatter(edges1, dst, zeros_big)
    h1, sum_n0, cnt_n, ts1, td1 = _run_node(
        True, h0, t10, t11, si0, si1, c0, c1, ohn, n0w1[0:128],
        n0w1[128:256], n0b1, n0w2, n0b2,
        [e1w1[0:128], e1w1[128:256], e1b1])
    virtual1 = _run_glob(sum_n0, cnt_n, sum_eb0, cnt_eb, virtual,
                         g0w1[0:128], g0w1[128:256], g0w1[256:384], g0b1,
                         g0w2, g0b2)

    gs1, gd1 = _sc_gather(src, dst, [ts1, td1], [0, 1])
    edges2, sum_eb1 = _run_edge(
        True, gs1, gd1, edges1, e1w1[256:384], e1w2, e1b2,
        wv=e1w1[384:512], virt=virtual1)
    t20, t21 = _sc_scatter(edges2, dst, zeros_big)
    h2, sum_n1 = _run_node(
        False, h1, t20, t21, t10, t11, c0, c1, ohn, n1w1[0:128],
        n1w1[128:256], n1b1, n1w2, n1b2, [n1w1[256:384], virtual1])
    virtual2 = _run_glob(sum_n1, cnt_n, sum_eb1, cnt_eb, virtual1,
                         g1w1[0:128], g1w1[128:256], g1w1[256:384], g1b1,
                         g1w2, g1b2)

    ret = _run_out(h2, [t20, t21], c0, c1, ohn, virtual2,
                   t, cond, ow1[0:128], ow1[128:256], ow1[256:384],
                   ow1[384:512], ow1[512:576], ob1, ow2, ob2)

    return ((h2, edges2, virtual2, edge_index, node_batch, lengths, t, cond),
            ret)


# speedup vs baseline: 7.3786x; 1.0996x over previous
import functools

import jax
import jax.numpy as jnp
from jax import lax
from jax.experimental import pallas as pl
from jax.experimental.pallas import tpu as pltpu
from jax.experimental.pallas import tpu_sc as plsc

NB = 16
NC, NS = 2, 16
NW = NC * NS
CHUNK = 128


def _silu(x):
    return x * jax.nn.sigmoid(x)


def _dot(a, b):
    return jnp.dot(a, b, preferred_element_type=jnp.float32)


def _dotT(a, b):
    return lax.dot_general(a, b, (((0,), (0,)), ((), ())),
                           preferred_element_type=jnp.float32)


def _full(shape):
    return pl.BlockSpec(shape, lambda i: (0,) * len(shape))


def _rows(r, c):
    return pl.BlockSpec((r, c), lambda i: (i, 0))


def _pre_body(nodes_ref, xp_ref, nb_ref, wn_ref, wx_ref, b1_ref, w2_ref,
              b2_ref, ws_ref, wd_ref, be_ref, h_ref, ts_ref, td_ref, oh_ref):
    r = nodes_ref.shape[0]
    hmid = _silu(_dot(nodes_ref[...], wn_ref[...])
                 + _dot(xp_ref[...], wx_ref[...]) + b1_ref[...])
    h = _dot(hmid, w2_ref[...]) + b2_ref[...]
    h_ref[...] = h
    cols = lax.broadcasted_iota(jnp.int32, (r, NB), 1)
    oh = (nb_ref[...] == cols).astype(jnp.float32)
    oh_ref[...] = oh
    z = jnp.zeros((r, 128 - 64 - NB), jnp.float32)
    ts_ref[...] = jnp.concatenate(
        [_dot(h, ws_ref[...]) + be_ref[...], oh, z], axis=1)
    td_ref[...] = jnp.concatenate(
        [_dot(h, wd_ref[...]), jnp.zeros((r, 64), jnp.float32)], axis=1)


def _run_pre(nodes, xp, nb2, wn, wx, b1, w2, b2, ws, wd, be):
    n = nodes.shape[0]
    r = 2000
    return pl.pallas_call(
        _pre_body,
        grid=(n // r,),
        in_specs=[_rows(r, 128), _rows(r, 128), _rows(r, 1),
                  _full(wn.shape), _full(wx.shape), _full(b1.shape),
                  _full(w2.shape), _full(b2.shape), _full(ws.shape),
                  _full(wd.shape), _full(be.shape)],
        out_specs=[_rows(r, 128), _rows(r, 128), _rows(r, 128), _rows(r, NB)],
        out_shape=[jax.ShapeDtypeStruct((n, 128), jnp.float32),
                   jax.ShapeDtypeStruct((n, 128), jnp.float32),
                   jax.ShapeDtypeStruct((n, 128), jnp.float32),
                   jax.ShapeDtypeStruct((n, NB), jnp.float32)],
    )(nodes, xp, nb2, wn, wx, b1, w2, b2, ws, wd, be)


def _edge_body(megnet, gs_ref, gd_ref, edges_ref, we_ref, w2_ref, b2_ref,
               *refs):
    if megnet:
        wv_ref, virt_ref = refs[0], refs[1]
        enew_ref, sum_eb_ref = refs[2], refs[3]
    else:
        enew_ref, sum_eb_ref, cnt_eb_ref = refs[:3]
    gs = gs_ref[...]
    oh = gs[:, 64:64 + NB]
    pre = gs[:, 0:64] + gd_ref[...][:, 0:64] + _dot(edges_ref[...], we_ref[...])
    if megnet:
        pre = pre + _dot(oh, _dot(virt_ref[...], wv_ref[...]))
    eo = _dot(_silu(pre), w2_ref[...]) + b2_ref[...]
    enew_ref[...] = edges_ref[...] + eo

    @pl.when(pl.program_id(0) == 0)
    def _():
        sum_eb_ref[...] = jnp.zeros_like(sum_eb_ref)
        if not megnet:
            cnt_eb_ref[...] = jnp.zeros_like(cnt_eb_ref)

    sum_eb_ref[...] += _dotT(oh, eo)
    if not megnet:
        cnt_eb_ref[...] += _dotT(oh, jnp.ones_like(eo))


def _run_edge(megnet, gs, gd, edges, we, w2, b2, wv=None, virt=None):
    e = edges.shape[0]
    r = 8000
    in_specs = [_rows(r, 128), _rows(r, 128), _rows(r, 128),
                _full(we.shape), _full(w2.shape), _full(b2.shape)]
    args = [gs, gd, edges, we, w2, b2]
    if megnet:
        in_specs += [_full(wv.shape), _full(virt.shape)]
        args += [wv, virt]
    out_specs = [_rows(r, 128), _full((NB, 128))]
    out_shape = [jax.ShapeDtypeStruct((e, 128), jnp.float32),
                 jax.ShapeDtypeStruct((NB, 128), jnp.float32)]
    if not megnet:
        out_specs.append(_full((NB, 128)))
        out_shape.append(jax.ShapeDtypeStruct((NB, 128), jnp.float32))
    return pl.pallas_call(
        functools.partial(_edge_body, megnet),
        grid=(e // r,),
        in_specs=in_specs,
        out_specs=out_specs,
        out_shape=out_shape,
    )(*args)


def _node_body(first, h_ref, pa0_ref, pa1_ref, pb0_ref, pb1_ref, c0_ref,
               c1_ref, oh_ref, wh_ref, wa_ref, b1_ref, w2_ref, b2_ref, *refs):
    if first:
        wsn_ref, wdn_ref, ben_ref = refs[0], refs[1], refs[2]
        hnew_ref, sumn_ref, cntn_ref, ts_ref, td_ref = refs[3:]
    else:
        wv_ref, virt_ref = refs[0], refs[1]
        hnew_ref, sumn_ref = refs[2], refs[3]
    oh = oh_ref[...]
    cnt = jnp.maximum(c0_ref[...] + c1_ref[...], 1.0)
    agg = ((pa0_ref[...] + pa1_ref[...])
           - (pb0_ref[...] + pb1_ref[...])) / cnt
    pre = _dot(h_ref[...], wh_ref[...]) + _dot(agg, wa_ref[...]) + b1_ref[...]
    if not first:
        pre = pre + _dot(oh, _dot(virt_ref[...], wv_ref[...]))
    nout = _dot(_silu(pre), w2_ref[...]) + b2_ref[...]
    hnew = h_ref[...] + nout
    hnew_ref[...] = hnew

    @pl.when(pl.program_id(0) == 0)
    def _():
        sumn_ref[...] = jnp.zeros_like(sumn_ref)
        if first:
            cntn_ref[...] = jnp.zeros_like(cntn_ref)

    sumn_ref[...] += _dotT(oh, nout)
    if first:
        cntn_ref[...] += _dotT(oh, jnp.ones_like(nout))
        r = oh.shape[0]
        z = jnp.zeros((r, 128 - 64 - NB), jnp.float32)
        ts_ref[...] = jnp.concatenate(
            [_dot(hnew, wsn_ref[...]) + ben_ref[...], oh, z], axis=1)
        td_ref[...] = jnp.concatenate(
            [_dot(hnew, wdn_ref[...]), jnp.zeros((r, 64), jnp.float32)],
            axis=1)


def _run_node(first, h, pa0, pa1, pb0, pb1, c0, c1, ohn, wh, wa, b1, w2, b2,
              extra):
    n = h.shape[0]
    r = 2000
    in_specs = [_rows(r, 128)] * 7 + [_rows(r, NB), _full(wh.shape),
                                      _full(wa.shape), _full(b1.shape),
                                      _full(w2.shape), _full(b2.shape)]
    args = [h, pa0, pa1, pb0, pb1, c0, c1, ohn, wh, wa, b1, w2, b2]
    for a in extra:
        in_specs.append(_full(a.shape))
        args.append(a)
    out_specs = [_rows(r, 128), _full((NB, 128))]
    out_shape = [jax.ShapeDtypeStruct((n, 128), jnp.float32),
                 jax.ShapeDtypeStruct((NB, 128), jnp.float32)]
    if first:
        out_specs += [_full((NB, 128)), _rows(r, 128), _rows(r, 128)]
        out_shape += [jax.ShapeDtypeStruct((NB, 128), jnp.float32),
                      jax.ShapeDtypeStruct((n, 128), jnp.float32),
                      jax.ShapeDtypeStruct((n, 128), jnp.float32)]
    return pl.pallas_call(
        functools.partial(_node_body, first),
        grid=(n // r,),
        in_specs=in_specs,
        out_specs=out_specs,
        out_shape=out_shape,
    )(*args)


def _glob_body(sumn_ref, cntn_ref, sume_ref, cnte_ref, virt_ref, wn_ref,
               we_ref, wv_ref, b1_ref, w2_ref, b2_ref, vnew_ref):
    nmean = sumn_ref[...] / jnp.maximum(cntn_ref[...], 1.0)
    emean = sume_ref[...] / jnp.maximum(cnte_ref[...], 1.0)
    hid = _silu(_dot(nmean, wn_ref[...]) + _dot(emean, we_ref[...])
                + _dot(virt_ref[...], wv_ref[...]) + b1_ref[...])
    vnew_ref[...] = virt_ref[...] + _dot(hid, w2_ref[...]) + b2_ref[...]


def _run_glob(sumn, cntn, sume, cnte, virt, wn, we, wv, b1, w2, b2):
    return pl.pallas_call(
        _glob_body,
        out_shape=jax.ShapeDtypeStruct((NB, 128), jnp.float32),
    )(sumn, cntn, sume, cnte, virt, wn, we, wv, b1, w2, b2)


def _out_body(h_ref, pa_ref, pb_ref, c0_ref, c1_ref, oh_ref, virt_ref, t_ref,
              cond_ref, wh_ref, wa_ref, wuv_ref, wut_ref, wuc_ref, b1_ref,
              w2_ref, b2_ref, ret_ref):
    cnt = jnp.maximum(c0_ref[...] + c1_ref[...], 1.0)
    agg = (pa_ref[...] + pb_ref[...]) / cnt
    uproj = (_dot(virt_ref[...], wuv_ref[...])
             + _dot(cond_ref[...], wuc_ref[...])
             + _dot(t_ref[...], wut_ref[...]))
    pre = (_dot(h_ref[...], wh_ref[...]) + _dot(agg, wa_ref[...])
           + _dot(oh_ref[...], uproj) + b1_ref[...])
    ret_ref[...] = _dot(_silu(pre), w2_ref[...]) + b2_ref[...]


def _run_out(h, parts, c0, c1, ohn, virt, t, cond, wh, wa, wuv, wut, wuc, b1,
             w2, b2):
    n = h.shape[0]
    r = 2000
    dout = b2.shape[1]
    in_specs = ([_rows(r, 128)] + [_rows(r, 128)] * 2
                + [_rows(r, 128), _rows(r, 128), _rows(r, NB)]
                + [_full(a.shape) for a in
                   (virt, t, cond, wh, wa, wuv, wut, wuc, b1, w2, b2)])
    return pl.pallas_call(
        _out_body,
        grid=(n // r,),
        in_specs=in_specs,
        out_specs=_rows(r, dout),
        out_shape=jax.ShapeDtypeStruct((n, dout), jnp.float32),
    )(h, *parts, c0, c1, ohn, virt, t, cond, wh, wa, wuv, wut, wuc, b1, w2, b2)


def _mesh():
    return plsc.VectorSubcoreMesh(core_axis_name="c", subcore_axis_name="s")


def _sc_gather(src, dst, tables, idx_sel):
    e = src.shape[0]
    nt = len(tables)
    n_chunks = e // CHUNK
    per_w = n_chunks // NW
    rem = n_chunks - per_w * NW
    assert per_w % 2 == 0
    out_type = tuple(jax.ShapeDtypeStruct((e, t.shape[1]), jnp.float32)
                     for t in tables)
    scratch = ([pltpu.VMEM((CHUNK,), jnp.int32)] * 4
               + [pltpu.VMEM((CHUNK, t.shape[1]), jnp.float32)
                  for t in tables for _ in range(2)]
               + [pltpu.SemaphoreType.DMA] * 6)

    @functools.partial(pl.kernel, out_type=out_type, mesh=_mesh(),
                       scratch_types=scratch)
    def k(src_hbm, dst_hbm, *refs):
        tabs = refs[:nt]
        outs = refs[nt:2 * nt]
        p = 2 * nt
        isrc = refs[p:p + 2]
        idst = refs[p + 2:p + 4]
        bufs = [refs[p + 4 + 2 * t:p + 6 + 2 * t] for t in range(nt)]
        semi = refs[p + 4 + 2 * nt:p + 6 + 2 * nt]
        semg = refs[p + 6 + 2 * nt:p + 8 + 2 * nt]
        semw = refs[p + 8 + 2 * nt:p + 10 + 2 * nt]
        cid = lax.axis_index("c")
        sid = lax.axis_index("s")
        wid = sid * NC + cid

        def off_of(i):
            return pl.multiple_of((i * NW + wid) * CHUNK, CHUNK)

        def issue_idx(i, b):
            off = off_of(i)
            pltpu.async_copy(src_hbm.at[pl.ds(off, CHUNK)], isrc[b], semi[b])
            pltpu.async_copy(dst_hbm.at[pl.ds(off, CHUNK)], idst[b], semi[b])

        def wait_idx(b):
            pltpu.make_async_copy(src_hbm.at[pl.ds(0, CHUNK)], isrc[b],
                                  semi[b]).wait()
            pltpu.make_async_copy(dst_hbm.at[pl.ds(0, CHUNK)], idst[b],
                                  semi[b]).wait()

        def run_gather(b):
            descs = []
            for t in range(nt):
                ib = isrc[b] if idx_sel[t] == 0 else idst[b]
                descs.append(pltpu.async_copy(tabs[t].at[ib], bufs[t][b],
                                              semg[b]))
            for d in descs:
                d.wait()

        def issue_wb(i, b):
            off = off_of(i)
            for t in range(nt):
                pltpu.async_copy(bufs[t][b], outs[t].at[pl.ds(off, CHUNK)],
                                 semw[b])

        def wait_wb(b):
            for t in range(nt):
                pltpu.make_async_copy(bufs[t][b],
                                      outs[t].at[pl.ds(0, CHUNK)],
                                      semw[b]).wait()

        issue_idx(0, 0)

        def body(j2, carry):
            for b in range(2):
                i = 2 * j2 + b

                @pl.when(i + 1 < per_w)
                def _():
                    issue_idx(i + 1, 1 - b)

                wait_idx(b)

                @pl.when(i >= 2)
                def _():
                    wait_wb(b)

                run_gather(b)
                issue_wb(i, b)
            return carry

        lax.fori_loop(0, per_w // 2, body, 0)
        wait_wb(0)
        wait_wb(1)
        if rem:
            @pl.when(wid < rem)
            def _():
                off = pl.multiple_of((per_w * NW + wid) * CHUNK, CHUNK)
                pltpu.sync_copy(src_hbm.at[pl.ds(off, CHUNK)], isrc[0])
                pltpu.sync_copy(dst_hbm.at[pl.ds(off, CHUNK)], idst[0])
                run_gather(0)
                for t in range(nt):
                    pltpu.sync_copy(bufs[t][0], outs[t].at[pl.ds(off, CHUNK)])

    return k(src, dst, *tables)


def _sc_scatter(v, idx, zeros_big):
    e, w = v.shape
    n = zeros_big.shape[0]
    n_chunks = e // CHUNK
    per_w = n_chunks // NW
    rem = n_chunks - per_w * NW
    rows = (n // NS) // 8 * 8
    tail = n - NS * rows
    out_type = tuple(jax.ShapeDtypeStruct((n, w), jnp.float32)
                     for _ in range(NC))
    assert per_w % 2 == 0
    scratch = [pltpu.VMEM((CHUNK,), jnp.int32)] * 2 + \
              [pltpu.VMEM((CHUNK, w), jnp.float32)] * 2 + \
              [pltpu.VMEM_SHARED((n, w), jnp.float32)] + \
              [pltpu.SemaphoreType.DMA] * 4

    @functools.partial(pl.kernel, out_type=out_type, mesh=_mesh(),
                       scratch_types=scratch)
    def k(v_hbm, idx_hbm, z_hbm, out0, out1, ibuf0, ibuf1, vbuf0, vbuf1, acc,
          seml0, seml1, sema0, sema1):
        ibuf = (ibuf0, ibuf1)
        vbuf = (vbuf0, vbuf1)
        seml = (seml0, seml1)
        sema = (sema0, sema1)
        cid = lax.axis_index("c")
        sid = lax.axis_index("s")
        wid = sid * NC + cid

        @pl.when(sid == 0)
        def _():
            pltpu.sync_copy(z_hbm, acc)

        plsc.subcore_barrier()

        def off_of(i):
            return pl.multiple_of((i * NW + wid) * CHUNK, CHUNK)

        def issue_load(i, b):
            off = off_of(i)
            pltpu.async_copy(idx_hbm.at[pl.ds(off, CHUNK)], ibuf[b], seml[b])
            pltpu.async_copy(v_hbm.at[pl.ds(off, CHUNK)], vbuf[b], seml[b])

        def wait_load(b):
            pltpu.make_async_copy(idx_hbm.at[pl.ds(0, CHUNK)], ibuf[b],
                                  seml[b]).wait()
            pltpu.make_async_copy(v_hbm.at[pl.ds(0, CHUNK)], vbuf[b],
                                  seml[b]).wait()

        issue_load(0, 0)

        def body(j2, carry):
            for b in range(2):
                i = 2 * j2 + b

                @pl.when(i + 1 < per_w)
                def _():
                    issue_load(i + 1, 1 - b)

                wait_load(b)
                pltpu.sync_copy(vbuf[b], acc.at[ibuf[b]], add=True)
            return carry

        lax.fori_loop(0, per_w // 2, body, 0)
        if rem:
            @pl.when(wid < rem)
            def _():
                off = pl.multiple_of((per_w * NW + wid) * CHUNK, CHUNK)
                pltpu.sync_copy(idx_hbm.at[pl.ds(off, CHUNK)], ibuf[0])
                pltpu.sync_copy(v_hbm.at[pl.ds(off, CHUNK)], vbuf[0])
                pltpu.sync_copy(vbuf[0], acc.at[ibuf[0]], add=True)
        plsc.subcore_barrier()

        r0 = pl.multiple_of(sid * rows, 8)

        @pl.when(cid == 0)
        def _():
            pltpu.sync_copy(acc.at[pl.ds(r0, rows)], out0.at[pl.ds(r0, rows)])

            @pl.when(sid == 0)
            def _():
                if tail:
                    pltpu.sync_copy(acc.at[pl.ds(NS * rows, tail)],
                                    out0.at[pl.ds(NS * rows, tail)])

        @pl.when(cid == 1)
        def _():
            pltpu.sync_copy(acc.at[pl.ds(r0, rows)], out1.at[pl.ds(r0, rows)])

            @pl.when(sid == 0)
            def _():
                if tail:
                    pltpu.sync_copy(acc.at[pl.ds(NS * rows, tail)],
                                    out1.at[pl.ds(NS * rows, tail)])

    return k(v, idx, zeros_big)


def _sc_counts(idx, zeros_big, ones_chunk):
    e = idx.shape[0]
    n, w = zeros_big.shape
    n_chunks = e // CHUNK
    per_w = n_chunks // NW
    rem = n_chunks - per_w * NW
    rows = (n // NS) // 8 * 8
    tail = n - NS * rows
    out_type = tuple(jax.ShapeDtypeStruct((n, w), jnp.float32)
                     for _ in range(NC))
    assert per_w % 2 == 0
    scratch = [pltpu.VMEM((CHUNK,), jnp.int32)] * 2 + \
              [pltpu.VMEM((CHUNK, w), jnp.float32)] + \
              [pltpu.VMEM_SHARED((n, w), jnp.float32)] + \
              [pltpu.SemaphoreType.DMA] * 4

    @functools.partial(pl.kernel, out_type=out_type, mesh=_mesh(),
                       scratch_types=scratch)
    def k(idx_hbm, z_hbm, ones_hbm, out0, out1, ibuf0, ibuf1, obuf, acc,
          seml0, seml1, sema0, sema1):
        ibuf = (ibuf0, ibuf1)
        seml = (seml0, seml1)
        sema = (sema0, sema1)
        cid = lax.axis_index("c")
        sid = lax.axis_index("s")
        wid = sid * NC + cid

        @pl.when(sid == 0)
        def _():
            pltpu.sync_copy(z_hbm, acc)

        pltpu.sync_copy(ones_hbm, obuf)
        plsc.subcore_barrier()

        def off_of(i):
            return pl.multiple_of((i * NW + wid) * CHUNK, CHUNK)

        def issue_load(i, b):
            pltpu.async_copy(idx_hbm.at[pl.ds(off_of(i), CHUNK)], ibuf[b],
                             seml[b])

        def wait_load(b):
            pltpu.make_async_copy(idx_hbm.at[pl.ds(0, CHUNK)], ibuf[b],
                                  seml[b]).wait()

        issue_load(0, 0)

        def body(j2, carry):
            for b in range(2):
                i = 2 * j2 + b

                @pl.when(i + 1 < per_w)
                def _():
                    issue_load(i + 1, 1 - b)

                wait_load(b)
                pltpu.sync_copy(obuf, acc.at[ibuf[b]], add=True)
            return carry

        lax.fori_loop(0, per_w // 2, body, 0)
        if rem:
            @pl.when(wid < rem)
            def _():
                off = pl.multiple_of((per_w * NW + wid) * CHUNK, CHUNK)
                pltpu.sync_copy(idx_hbm.at[pl.ds(off, CHUNK)], ibuf[0])
                pltpu.sync_copy(obuf, acc.at[ibuf[0]], add=True)
        plsc.subcore_barrier()

        r0 = pl.multiple_of(sid * rows, 8)

        @pl.when(cid == 0)
        def _():
            pltpu.sync_copy(acc.at[pl.ds(r0, rows)], out0.at[pl.ds(r0, rows)])

            @pl.when(sid == 0)
            def _():
                if tail:
                    pltpu.sync_copy(acc.at[pl.ds(NS * rows, tail)],
                                    out0.at[pl.ds(NS * rows, tail)])

        @pl.when(cid == 1)
        def _():
            pltpu.sync_copy(acc.at[pl.ds(r0, rows)], out1.at[pl.ds(r0, rows)])

            @pl.when(sid == 0)
            def _():
                if tail:
                    pltpu.sync_copy(acc.at[pl.ds(NS * rows, tail)],
                                    out1.at[pl.ds(NS * rows, tail)])

    return k(idx, zeros_big, ones_chunk)


def kernel(nodes, x, edges, virtual, edge_index, node_batch, lengths, t, cond,
           params):
    n = nodes.shape[0]
    src = edge_index[0]
    dst = edge_index[1]

    (w1p, b1p), (w2p, b2p) = params["pe"]
    wn_p = w1p[:128]
    wx_p = jnp.pad(w1p[128:], ((0, 128 - (w1p.shape[0] - 128)), (0, 0)))
    xp = jnp.pad(x, ((0, 0), (0, 128 - x.shape[1])))
    nb2 = node_batch.reshape(n, 1)

    def lin(layer):
        w, b = layer
        return w, b.reshape(1, -1)

    e0w1, e0b1 = lin(params["mpnn0"]["edge"][0])
    e0w2, e0b2 = lin(params["mpnn0"]["edge"][1])
    n0w1, n0b1 = lin(params["mpnn0"]["node"][0])
    n0w2, n0b2 = lin(params["mpnn0"]["node"][1])
    g0w1, g0b1 = lin(params["mpnn0"]["glob"][0])
    g0w2, g0b2 = lin(params["mpnn0"]["glob"][1])
    e1w1, e1b1 = lin(params["mpnn1"]["edge"][0])
    e1w2, e1b2 = lin(params["mpnn1"]["edge"][1])
    n1w1, n1b1 = lin(params["mpnn1"]["node"][0])
    n1w2, n1b2 = lin(params["mpnn1"]["node"][1])
    g1w1, g1b1 = lin(params["mpnn1"]["glob"][0])
    g1w2, g1b2 = lin(params["mpnn1"]["glob"][1])
    ow1, ob1 = lin(params["out"][0])
    ow2, ob2 = lin(params["out"][1])

    zeros_big = jnp.zeros((n, 128), jnp.float32)
    ones_chunk = jnp.ones((CHUNK, 128), jnp.float32)

    h0, ts0, td0, ohn = _run_pre(
        nodes, xp, nb2, wn_p, wx_p, b1p.reshape(1, -1), w2p,
        b2p.reshape(1, -1), e0w1[0:128], e0w1[128:256], e0b1)

    c0, c1 = _sc_counts(dst, zeros_big, ones_chunk)
    si0, si1 = _sc_scatter(edges, dst, zeros_big)

    gs0, gd0 = _sc_gather(src, dst, [ts0, td0], [0, 1])
    edges1, sum_eb0, cnt_eb = _run_edge(
        False, gs0, gd0, edges, e0w1[256:384], e0w2, e0b2)
    t10, t11 = _sc_scatter(edges1, dst, zeros_big)
    h1, sum_n0, cnt_n, ts1, td1 = _run_node(
        True, h0, t10, t11, si0, si1, c0, c1, ohn, n0w1[0:128],
        n0w1[128:256], n0b1, n0w2, n0b2,
        [e1w1[0:128], e1w1[128:256], e1b1])
    virtual1 = _run_glob(sum_n0, cnt_n, sum_eb0, cnt_eb, virtual,
                         g0w1[0:128], g0w1[128:256], g0w1[256:384], g0b1,
                         g0w2, g0b2)

    gs1, gd1 = _sc_gather(src, dst, [ts1, td1], [0, 1])
    edges2, sum_eb1 = _run_edge(
        True, gs1, gd1, edges1, e1w1[256:384], e1w2, e1b2,
        wv=e1w1[384:512], virt=virtual1)
    t20, t21 = _sc_scatter(edges2, dst, zeros_big)
    h2, sum_n1 = _run_node(
        False, h1, t20, t21, t10, t11, c0, c1, ohn, n1w1[0:128],
        n1w1[128:256], n1b1, n1w2, n1b2, [n1w1[256:384], virtual1])
    virtual2 = _run_glob(sum_n1, cnt_n, sum_eb1, cnt_eb, virtual1,
                         g1w1[0:128], g1w1[128:256], g1w1[256:384], g1b1,
                         g1w2, g1b2)

    ret = _run_out(h2, [t20, t21], c0, c1, ohn, virtual2,
                   t, cond, ow1[0:128], ow1[128:256], ow1[256:384],
                   ow1[384:512], ow1[512:576], ob1, ow2, ob2)

    return ((h2, edges2, virtual2, edge_index, node_batch, lengths, t, cond),
            ret)
